# Initial kernel scaffold; baseline (speedup 1.0000x reference)
import jax
import jax.numpy as jnp
from jax.experimental import pallas as pl


def kernel(feature, edge_index, label, train_nodes, teacher_logit, params):
    raise NotImplementedError("write your pallas kernel here")


# speedup vs baseline: 1.5748x; 1.5748x over previous
import functools

import jax
import jax.numpy as jnp
from jax import lax
from jax.experimental import pallas as pl
from jax.experimental.pallas import tpu as pltpu
from jax.experimental.pallas import tpu_sc as plsc

TAU = 1.0
LAMBDA1 = 0.5
LAMBDA2 = 0.3
EPS = 1e-5

NC = 2
NS = 16
NQ = 2


def _colstats_body(x_ref, s_ref, q_ref):
    i = pl.program_id(0)
    x = x_ref[...]
    s = jnp.sum(x, axis=0, keepdims=True)
    q = jnp.sum(x * x, axis=0, keepdims=True)

    @pl.when(i == 0)
    def _():
        s_ref[...] = s
        q_ref[...] = q

    @pl.when(i > 0)
    def _():
        s_ref[...] += s
        q_ref[...] += q


def _colstats(x, blk):
    n, d = x.shape
    return pl.pallas_call(
        _colstats_body,
        grid=(n // blk,),
        in_specs=[pl.BlockSpec((blk, d), lambda i: (i, 0))],
        out_specs=[pl.BlockSpec((1, d), lambda i: (0, 0)),
                   pl.BlockSpec((1, d), lambda i: (0, 0))],
        out_shape=[jax.ShapeDtypeStruct((1, d), jnp.float32),
                   jax.ShapeDtypeStruct((1, d), jnp.float32)],
    )(x)


def _bnapply_body(x_ref, s_ref, q_ref, g_ref, b_ref, *h_refs, n):
    mu = s_ref[...] / n
    var = q_ref[...] / n - mu * mu
    sc = g_ref[...] * lax.rsqrt(var + EPS)
    sh = b_ref[...] - mu * sc
    h = x_ref[...] * sc + sh
    dq = h.shape[1] // NQ
    for j, h_ref in enumerate(h_refs):
        h_ref[...] = h[:, j * dq:(j + 1) * dq]


def _bnapply(x, s, q, g, b, blk):
    n, d = x.shape
    dq = d // NQ
    return pl.pallas_call(
        functools.partial(_bnapply_body, n=float(n)),
        grid=(n // blk,),
        in_specs=[pl.BlockSpec((blk, d), lambda i: (i, 0)),
                  pl.BlockSpec((1, d), lambda i: (0, 0)),
                  pl.BlockSpec((1, d), lambda i: (0, 0)),
                  pl.BlockSpec((1, d), lambda i: (0, 0)),
                  pl.BlockSpec((1, d), lambda i: (0, 0))],
        out_specs=[pl.BlockSpec((blk, dq), lambda i: (i, 0))] * NQ,
        out_shape=[jax.ShapeDtypeStruct((n, dq), jnp.float32)] * NQ,
    )(x, s, q, g, b)


def _sc_agg_build(n, e, dh):
    K = 128
    EW = e // NS
    NCH = EW // K
    ZBR = 40
    HALF = n // 2
    NZ = HALF // ZBR
    ZPER = -(-NZ // NS)
    JUNK = HALF
    mesh = plsc.VectorSubcoreMesh(core_axis_name="c", subcore_axis_name="s")

    @functools.partial(
        pl.kernel,
        mesh=mesh,
        out_type=[jax.ShapeDtypeStruct((n, dh), jnp.float32)] * NQ,
        scratch_types=[
            pltpu.VMEM((K,), jnp.int32),
            pltpu.VMEM((K,), jnp.int32),
            pltpu.VMEM((K,), jnp.int32),
            pltpu.VMEM((K, dh), jnp.float32),
            pltpu.VMEM((ZBR, dh), jnp.float32),
            pltpu.VMEM((ZBR, dh), jnp.float32),
            pltpu.VMEM_SHARED((HALF + 8, dh), jnp.float32),
            pltpu.SemaphoreType.DMA,
        ],
    )
    def sc_agg(h0, h1, src, dst, agg0, agg1,
               src_v, dst_v, idx_v, rows_v, zero_v, bounce_v, acc, sem):
        cid = lax.axis_index("c")
        sid = lax.axis_index("s")
        z16f = jnp.zeros((16,), jnp.float32)

        def zrow(i, _):
            r = i // (dh // 16)
            c16 = i % (dh // 16)
            zero_v[r, pl.ds(c16 * 16, 16)] = z16f
            return _
        lax.fori_loop(0, ZBR * (dh // 16), zrow, None)

        def zacc(t, _):
            j = sid + t * NS

            @pl.when(j < NZ)
            def _():
                pltpu.sync_copy(zero_v, acc.at[pl.ds(j * ZBR, ZBR)])
            return _

        def scan_edges(h_ref, lo):
            def chunk(c, _):
                base = sid * EW + c * K
                pltpu.sync_copy(src.at[pl.ds(base, K)], src_v)
                pltpu.sync_copy(dst.at[pl.ds(base, K)], dst_v)

                def adj(j, _):
                    d16 = dst_v[pl.ds(j * 16, 16)]
                    rel = d16 - lo
                    ok = jnp.logical_and(rel >= 0, rel < HALF)
                    idx_v[pl.ds(j * 16, 16)] = jnp.where(ok, rel, JUNK)
                    return _
                lax.fori_loop(0, K // 16, adj, None)
                pltpu.async_copy(h_ref.at[src_v], rows_v, sem).wait()
                pltpu.sync_copy(rows_v, acc.at[idx_v], add=True)
                return _
            lax.fori_loop(0, NCH, chunk, None)

        def flush(lo, agg_ref):
            def go(t, _):
                j = sid + t * NS

                @pl.when(j < NZ)
                def _():
                    pltpu.sync_copy(acc.at[pl.ds(j * ZBR, ZBR)], bounce_v)
                    pltpu.sync_copy(bounce_v,
                                    agg_ref.at[pl.ds(lo + j * ZBR, ZBR)])
                return _
            lax.fori_loop(0, ZPER, go, None)

        lax.fori_loop(0, ZPER, zacc, None)
        plsc.subcore_barrier()

        @pl.when(cid == 0)
        def _():
            scan_edges(h0, 0)

        @pl.when(cid == 1)
        def _():
            scan_edges(h1, 0)

        plsc.subcore_barrier()

        @pl.when(cid == 0)
        def _():
            flush(0, agg0)

        @pl.when(cid == 1)
        def _():
            flush(0, agg1)

        plsc.subcore_barrier()

        lax.fori_loop(0, ZPER, zacc, None)
        plsc.subcore_barrier()

        @pl.when(cid == 0)
        def _():
            scan_edges(h0, HALF)

        @pl.when(cid == 1)
        def _():
            scan_edges(h1, HALF)

        plsc.subcore_barrier()

        @pl.when(cid == 0)
        def _():
            flush(HALF, agg0)

        @pl.when(cid == 1)
        def _():
            flush(HALF, agg1)

    return sc_agg


def _hist_body(d_ref, out_ref, *, ngrid, recip):
    i = pl.program_id(0)
    d = d_ref[0]
    cols = lax.broadcasted_iota(jnp.int32, (d.shape[0], 128), 1)
    hi = lax.shift_right_logical(d, 7)
    lo = jnp.bitwise_and(d, 127)
    a = (hi == cols).astype(jnp.float32)
    b = (lo == cols).astype(jnp.float32)
    m = lax.dot_general(a, b, (((0,), (0,)), ((), ())),
                        preferred_element_type=jnp.float32)

    @pl.when(i == 0)
    def _():
        out_ref[...] = m

    @pl.when(i > 0)
    def _():
        out_ref[...] += m

    if recip:
        @pl.when(i == ngrid - 1)
        def _():
            out_ref[...] = 1.0 / jnp.maximum(out_ref[...], 1.0)


def _hist(idx, blk, recip):
    (e,) = idx.shape
    ngrid = e // blk
    idx3 = idx.reshape(ngrid, blk, 1)
    return pl.pallas_call(
        functools.partial(_hist_body, ngrid=ngrid, recip=recip),
        grid=(ngrid,),
        in_specs=[pl.BlockSpec((1, blk, 1), lambda i: (i, 0, 0))],
        out_specs=pl.BlockSpec((128, 128), lambda i: (0, 0)),
        out_shape=jax.ShapeDtypeStruct((128, 128), jnp.float32),
    )(idx3)


def _mlp_tail(xq, w0_ref, b0_ref, w1_ref, b1_ref, w2_ref, b2_ref,
              out_ref, s_ref, q_ref):
    i = pl.program_id(0)
    w0 = w0_ref[...]
    dq = xq[0].shape[1]
    y = b0_ref[...].astype(jnp.float32)
    for j, x in enumerate(xq):
        y = y + jnp.dot(x, w0[j * dq:(j + 1) * dq, :],
                        preferred_element_type=jnp.float32)
    y = jnp.maximum(y, 0.0)
    y = jnp.dot(y, w1_ref[...], preferred_element_type=jnp.float32) + b1_ref[...]
    y = jnp.maximum(y, 0.0)
    y = jnp.dot(y, w2_ref[...], preferred_element_type=jnp.float32) + b2_ref[...]
    y = jnp.maximum(y, 0.0)
    out_ref[...] = y
    s = jnp.sum(y, axis=0, keepdims=True)
    q = jnp.sum(y * y, axis=0, keepdims=True)

    @pl.when(i == 0)
    def _():
        s_ref[...] = s
        q_ref[...] = q

    @pl.when(i > 0)
    def _():
        s_ref[...] += s
        q_ref[...] += q


def _mlp_hom_body(a0, a1, di_ref, w0, b0, w1, b1, w2, b2,
                  out_ref, s_ref, q_ref):
    di = di_ref[...]
    xq = [a[...] * di for a in (a0, a1)]
    _mlp_tail(xq, w0, b0, w1, b1, w2, b2, out_ref, s_ref, q_ref)


def _mlp_het_body(f0, f1, a0, a1, di_ref, w0, b0, w1, b1,
                  w2, b2, out_ref, s_ref, q_ref):
    di = di_ref[...]
    xq = [f[...] - a[...] * di for f, a in zip((f0, f1), (a0, a1))]
    _mlp_tail(xq, w0, b0, w1, b1, w2, b2, out_ref, s_ref, q_ref)


def _mlp_single_body(h0, h1, w0, b0, w1, b1, w2, b2,
                     out_ref, s_ref, q_ref):
    _mlp_tail([h0[...], h1[...]],
              w0, b0, w1, b1, w2, b2, out_ref, s_ref, q_ref)


def _mlp_call(body, row_args, row_specs, dinv_col, p, blk):
    n = row_args[0].shape[0]
    h = p["W1"].shape[0]
    args = list(row_args)
    specs = list(row_specs)
    if dinv_col is not None:
        args.append(dinv_col)
        specs.append(pl.BlockSpec((blk, 1), lambda i: (i, 0)))
    d0 = p["W0"].shape[0]
    args += [p["W0"], p["b0"].reshape(1, h), p["W1"], p["b1"].reshape(1, h),
             p["W2"], p["b2"].reshape(1, h)]
    specs += [pl.BlockSpec((d0, h), lambda i: (0, 0)),
              pl.BlockSpec((1, h), lambda i: (0, 0)),
              pl.BlockSpec((h, h), lambda i: (0, 0)),
              pl.BlockSpec((1, h), lambda i: (0, 0)),
              pl.BlockSpec((h, h), lambda i: (0, 0)),
              pl.BlockSpec((1, h), lambda i: (0, 0))]
    return pl.pallas_call(
        body,
        grid=(n // blk,),
        in_specs=specs,
        out_specs=[pl.BlockSpec((blk, h), lambda i: (i, 0)),
                   pl.BlockSpec((1, h), lambda i: (0, 0)),
                   pl.BlockSpec((1, h), lambda i: (0, 0))],
        out_shape=[jax.ShapeDtypeStruct((n, h), jnp.float32),
                   jax.ShapeDtypeStruct((1, h), jnp.float32),
                   jax.ShapeDtypeStruct((1, h), jnp.float32)],
    )(*args)


def _combine_body(lh_ref, le_ref, ls_ref, sh_ref, qh_ref, se_ref, qe_ref,
                  ss_ref, qs_ref, gh_ref, bh_ref, ge_ref, be_ref, gs_ref,
                  bs_ref, f_ref, gw_ref, gb_ref, fw_ref, fb_ref, t_ref,
                  lab_ref, cnt_ref, logit_ref, kl_ref, ce_ref, s0_ref,
                  s1_ref, s2_ref, loss_ref, *, n, nt, ngrid):
    i = pl.program_id(0)

    def norm(l_ref, s_ref, q_ref, g_ref, b_ref):
        mu = s_ref[...] / n
        var = q_ref[...] / n - mu * mu
        sc = g_ref[...] * lax.rsqrt(var + EPS)
        return l_ref[...] * sc + (b_ref[...] - mu * sc)

    xh = norm(lh_ref, sh_ref, qh_ref, gh_ref, bh_ref)
    xe = norm(le_ref, se_ref, qe_ref, ge_ref, be_ref)
    xs = norm(ls_ref, ss_ref, qs_ref, gs_ref, bs_ref)

    z = (jnp.dot(f_ref[...], gw_ref[...], preferred_element_type=jnp.float32)
         + gb_ref[...])
    zm = jnp.max(z, axis=1, keepdims=True)
    ez = jnp.exp(z - zm)
    cf = ez / jnp.sum(ez, axis=1, keepdims=True)
    c0 = cf[:, 0:1]
    c1 = cf[:, 1:2]
    c2 = cf[:, 2:3]
    m = jnp.minimum(c0, jnp.minimum(c1, c2))
    e2 = c2 <= m
    e1 = jnp.logical_and(jnp.logical_not(e2), c1 <= m)
    e0 = jnp.logical_and(jnp.logical_not(e2), jnp.logical_not(e1))
    mk0 = jnp.where(e0, c0 * -100000.0, c0)
    mk1 = jnp.where(e1, c1 * -100000.0, c1)
    mk2 = jnp.where(e2, c2 * -100000.0, c2)
    mx = jnp.maximum(mk0, jnp.maximum(mk1, mk2))
    x0 = jnp.exp(mk0 - mx)
    x1 = jnp.exp(mk1 - mx)
    x2 = jnp.exp(mk2 - mx)
    zs = x0 + x1 + x2
    w0 = x0 / zs
    w1 = x1 / zs
    w2 = x2 / zs

    emb = w0 * xh + w1 * xe + w2 * xs
    lr = (jnp.dot(emb, fw_ref[...], preferred_element_type=jnp.float32)
          + fb_ref[...])
    lm = jnp.max(lr, axis=1, keepdims=True)
    el = jnp.exp(lr - lm)
    sm = el / jnp.sum(el, axis=1, keepdims=True)
    logit_ref[...] = sm

    t = sm / TAU
    kl = jnp.sum(t * (jnp.log(t) - t_ref[...] / TAU)).reshape(1, 1)

    mx2 = jnp.max(sm, axis=1, keepdims=True)
    logp = sm - mx2 - jnp.log(jnp.sum(jnp.exp(sm - mx2), axis=1, keepdims=True))
    lanes = lax.broadcasted_iota(jnp.int32, sm.shape, 1)
    pick = jnp.sum(jnp.where(lanes == lab_ref[...], logp, 0.0), axis=1,
                   keepdims=True)
    ce = (-jnp.sum(cnt_ref[...] * pick)).reshape(1, 1)

    s0 = jnp.sum(w0).reshape(1, 1)
    s1 = jnp.sum(w1).reshape(1, 1)
    s2 = jnp.sum(w2).reshape(1, 1)

    @pl.when(i == 0)
    def _():
        kl_ref[...] = kl
        ce_ref[...] = ce
        s0_ref[...] = s0
        s1_ref[...] = s1
        s2_ref[...] = s2

    @pl.when(i > 0)
    def _():
        kl_ref[...] += kl
        ce_ref[...] += ce
        s0_ref[...] += s0
        s1_ref[...] += s1
        s2_ref[...] += s2

    @pl.when(i == ngrid - 1)
    def _():
        loss1 = kl_ref[...] / n
        loss2 = ce_ref[...] / nt
        third = 1.0 / 3.0
        aux = (jnp.abs(s0_ref[...] / n - third)
               + jnp.abs(s1_ref[...] / n - third)
               + jnp.abs(s2_ref[...] / n - third))
        loss_ref[...] = (LAMBDA1 * loss1 + (1.0 - LAMBDA1) * loss2
                         + LAMBDA2 * aux)


def _combine(lh, le, ls, stats, bns, feature, gwp, gbp, fw, fb, teacher,
             lab_col, cnt_col, nt, blk):
    n, h = lh.shape
    c = fw.shape[1]
    d = feature.shape[1]
    ngrid = n // blk
    sh, qh, se, qe, ss, qs = stats
    gh, bh, ge, be, gs, bs = bns
    row = lambda w: pl.BlockSpec((blk, w), lambda i: (i, 0))
    one = lambda w: pl.BlockSpec((1, w), lambda i: (0, 0))
    scl = lambda: pl.BlockSpec((1, 1), lambda i: (0, 0))
    return pl.pallas_call(
        functools.partial(_combine_body, n=float(n), nt=float(nt),
                          ngrid=ngrid),
        grid=(ngrid,),
        in_specs=[row(h), row(h), row(h),
                  one(h), one(h), one(h), one(h), one(h), one(h),
                  one(h), one(h), one(h), one(h), one(h), one(h),
                  row(d), pl.BlockSpec((d, 128), lambda i: (0, 0)), one(128),
                  pl.BlockSpec((h, c), lambda i: (0, 0)), one(c),
                  row(c), row(1), row(1)],
        out_specs=[row(c), scl(), scl(), scl(), scl(), scl(), scl()],
        out_shape=[jax.ShapeDtypeStruct((n, c), jnp.float32)]
                  + [jax.ShapeDtypeStruct((1, 1), jnp.float32)] * 6,
    )(lh, le, ls, sh, qh, se, qe, ss, qs, gh, bh, ge, be, gs, bs,
      feature, gwp, gbp, fw, fb, teacher, lab_col, cnt_col)


def kernel(feature, edge_index, label, train_nodes, teacher_logit, params):
    n, d = feature.shape
    e = edge_index.shape[1]
    nt = train_nodes.shape[0]
    h = params["fin_W"].shape[0]
    c = params["fin_W"].shape[1]
    dq = d // NQ

    fsum, fsq = _colstats(feature, 2000)
    hq = _bnapply(feature, fsum, fsq,
                  params["bn_in_g"].reshape(1, d),
                  params["bn_in_b"].reshape(1, d), 2000)

    KCH = 4096
    ep = -(-e // KCH) * KCH
    src_p = jnp.concatenate(
        [edge_index[0], jnp.zeros((ep - e,), jnp.int32)])
    dst_p = jnp.concatenate(
        [edge_index[1], jnp.full((ep - e,), n, jnp.int32)])
    ntp = -(-nt // 128) * 128
    tn_p = jnp.concatenate([train_nodes, jnp.full((ntp - nt,), n, jnp.int32)])

    sc_agg = _sc_agg_build(n, ep, dq)
    agg0, agg1 = sc_agg(hq[0], hq[1], src_p, dst_p)
    aggq = [agg0, agg1]
    dinv_col = _hist(dst_p, 4096, True).reshape(16384, 1)[:n]
    cnt_col = _hist(tn_p, ntp, False).reshape(16384, 1)[:n]

    MB = 1000
    aspec = [pl.BlockSpec((MB, dq), lambda i: (i, 0)) for _ in range(NQ)]
    fq = [feature[:, j * dq:(j + 1) * dq] for j in range(NQ)]

    lh, sh_, qh_ = _mlp_call(_mlp_hom_body, aggq, aspec, dinv_col,
                             params["hom"], MB)
    le, se_, qe_ = _mlp_call(_mlp_het_body, fq + aggq,
                             aspec + aspec, dinv_col, params["het"], MB)
    ls, ss_, qs_ = _mlp_call(_mlp_single_body, list(hq), aspec, None,
                             params["single"], MB)

    gwp = jnp.zeros((d, 128), jnp.float32).at[:, :3].set(params["gate_W"])
    gbp = jnp.full((1, 128), -1e30, jnp.float32).at[0, :3].set(params["gate_b"])
    stats = (sh_, qh_, se_, qe_, ss_, qs_)
    bns = (params["hom"]["bn_g"].reshape(1, h), params["hom"]["bn_b"].reshape(1, h),
           params["het"]["bn_g"].reshape(1, h), params["het"]["bn_b"].reshape(1, h),
           params["single"]["bn_g"].reshape(1, h), params["single"]["bn_b"].reshape(1, h))
    logit, _kl, _ce, _s0, _s1, _s2, loss = _combine(
        lh, le, ls, stats, bns, feature, gwp, gbp,
        params["fin_W"], params["fin_b"].reshape(1, c), teacher_logit,
        label.reshape(n, 1), cnt_col, nt, 400)
    return logit, loss.reshape(())


# speedup vs baseline: 2.4851x; 1.5780x over previous
import functools

import jax
import jax.numpy as jnp
from jax import lax
from jax.experimental import pallas as pl
from jax.experimental.pallas import tpu as pltpu
from jax.experimental.pallas import tpu_sc as plsc

TAU = 1.0
LAMBDA1 = 0.5
LAMBDA2 = 0.3
EPS = 1e-5

NC = 2
NS = 16
NQ = 2


def _colstats_body(x_ref, s_ref, q_ref):
    i = pl.program_id(0)
    x = x_ref[...]
    s = jnp.sum(x, axis=0, keepdims=True)
    q = jnp.sum(x * x, axis=0, keepdims=True)

    @pl.when(i == 0)
    def _():
        s_ref[...] = s
        q_ref[...] = q

    @pl.when(i > 0)
    def _():
        s_ref[...] += s
        q_ref[...] += q


def _colstats(x, blk):
    n, d = x.shape
    return pl.pallas_call(
        _colstats_body,
        grid=(n // blk,),
        in_specs=[pl.BlockSpec((blk, d), lambda i: (i, 0))],
        out_specs=[pl.BlockSpec((1, d), lambda i: (0, 0)),
                   pl.BlockSpec((1, d), lambda i: (0, 0))],
        out_shape=[jax.ShapeDtypeStruct((1, d), jnp.float32),
                   jax.ShapeDtypeStruct((1, d), jnp.float32)],
    )(x)


def _bnapply_body(x_ref, s_ref, q_ref, g_ref, b_ref, *h_refs, n):
    mu = s_ref[...] / n
    var = q_ref[...] / n - mu * mu
    sc = g_ref[...] * lax.rsqrt(var + EPS)
    sh = b_ref[...] - mu * sc
    h = x_ref[...] * sc + sh
    dq = h.shape[1] // NQ
    for j, h_ref in enumerate(h_refs):
        h_ref[...] = h[:, j * dq:(j + 1) * dq]


def _bnapply(x, s, q, g, b, blk):
    n, d = x.shape
    dq = d // NQ
    return pl.pallas_call(
        functools.partial(_bnapply_body, n=float(n)),
        grid=(n // blk,),
        in_specs=[pl.BlockSpec((blk, d), lambda i: (i, 0)),
                  pl.BlockSpec((1, d), lambda i: (0, 0)),
                  pl.BlockSpec((1, d), lambda i: (0, 0)),
                  pl.BlockSpec((1, d), lambda i: (0, 0)),
                  pl.BlockSpec((1, d), lambda i: (0, 0))],
        out_specs=[pl.BlockSpec((blk, dq), lambda i: (i, 0))] * NQ,
        out_shape=[jax.ShapeDtypeStruct((n, dq), jnp.float32)] * NQ,
    )(x, s, q, g, b)


def _sc_agg_build(n, e, dh):
    K = 128
    EW = e // NS
    NCH = EW // K
    ZBR = 8
    NZ = n // ZBR
    ZPER = -(-NZ // NS)
    mesh = plsc.VectorSubcoreMesh(core_axis_name="c", subcore_axis_name="s")

    @functools.partial(
        pl.kernel,
        mesh=mesh,
        out_type=[jax.ShapeDtypeStruct((n, dh), jnp.float32)] * NQ,
        scratch_types=[
            pltpu.VMEM((K,), jnp.int32),
            pltpu.VMEM((K,), jnp.int32),
            pltpu.VMEM((K, dh), jnp.float32),
            pltpu.VMEM((ZBR, dh), jnp.float32),
            pltpu.VMEM((ZBR, dh), jnp.float32),
            pltpu.VMEM_SHARED((n + 8, dh), jnp.float32),
            pltpu.SemaphoreType.DMA,
        ],
    )
    def sc_agg(h0, h1, src, dst, agg0, agg1,
               src_v, dst_v, rows_v, zero_v, bounce_v, acc, sem):
        cid = lax.axis_index("c")
        sid = lax.axis_index("s")
        z16f = jnp.zeros((16,), jnp.float32)

        def zrow(i, _):
            r = i // (dh // 16)
            c16 = i % (dh // 16)
            zero_v[r, pl.ds(c16 * 16, 16)] = z16f
            return _
        lax.fori_loop(0, ZBR * (dh // 16), zrow, None)

        def zacc(t, _):
            j = sid + t * NS

            @pl.when(j < NZ)
            def _():
                pltpu.sync_copy(zero_v, acc.at[pl.ds(j * ZBR, ZBR)])
            return _
        lax.fori_loop(0, ZPER, zacc, None)
        plsc.subcore_barrier()

        def scan_edges(h_ref):
            def chunk(c, _):
                base = sid * EW + c * K
                pltpu.sync_copy(src.at[pl.ds(base, K)], src_v)
                pltpu.sync_copy(dst.at[pl.ds(base, K)], dst_v)
                pltpu.async_copy(h_ref.at[src_v], rows_v, sem).wait()
                pltpu.sync_copy(rows_v, acc.at[dst_v], add=True)
                return _
            lax.fori_loop(0, NCH, chunk, None)

        @pl.when(cid == 0)
        def _():
            scan_edges(h0)

        @pl.when(cid == 1)
        def _():
            scan_edges(h1)

        plsc.subcore_barrier()

        def flush(agg_ref):
            def go(t, _):
                j = sid + t * NS

                @pl.when(j < NZ)
                def _():
                    pltpu.sync_copy(acc.at[pl.ds(j * ZBR, ZBR)], bounce_v)
                    pltpu.sync_copy(bounce_v, agg_ref.at[pl.ds(j * ZBR, ZBR)])
                return _
            lax.fori_loop(0, ZPER, go, None)

        @pl.when(cid == 0)
        def _():
            flush(agg0)

        @pl.when(cid == 1)
        def _():
            flush(agg1)

    return sc_agg


def _hist_body(d_ref, out_ref, *, ngrid, recip):
    i = pl.program_id(0)
    d = d_ref[0]
    cols = lax.broadcasted_iota(jnp.int32, (d.shape[0], 128), 1)
    hi = lax.shift_right_logical(d, 7)
    lo = jnp.bitwise_and(d, 127)
    a = (hi == cols).astype(jnp.float32)
    b = (lo == cols).astype(jnp.float32)
    m = lax.dot_general(a, b, (((0,), (0,)), ((), ())),
                        preferred_element_type=jnp.float32)

    @pl.when(i == 0)
    def _():
        out_ref[...] = m

    @pl.when(i > 0)
    def _():
        out_ref[...] += m

    if recip:
        @pl.when(i == ngrid - 1)
        def _():
            out_ref[...] = 1.0 / jnp.maximum(out_ref[...], 1.0)


def _hist(idx, blk, recip):
    (e,) = idx.shape
    ngrid = e // blk
    idx3 = idx.reshape(ngrid, blk, 1)
    return pl.pallas_call(
        functools.partial(_hist_body, ngrid=ngrid, recip=recip),
        grid=(ngrid,),
        in_specs=[pl.BlockSpec((1, blk, 1), lambda i: (i, 0, 0))],
        out_specs=pl.BlockSpec((128, 128), lambda i: (0, 0)),
        out_shape=jax.ShapeDtypeStruct((128, 128), jnp.float32),
    )(idx3)


def _mlp_tail(xq, w0_ref, b0_ref, w1_ref, b1_ref, w2_ref, b2_ref,
              out_ref, s_ref, q_ref):
    i = pl.program_id(0)
    w0 = w0_ref[...]
    dq = xq[0].shape[1]
    y = b0_ref[...].astype(jnp.float32)
    for j, x in enumerate(xq):
        y = y + jnp.dot(x, w0[j * dq:(j + 1) * dq, :],
                        preferred_element_type=jnp.float32)
    y = jnp.maximum(y, 0.0)
    y = jnp.dot(y, w1_ref[...], preferred_element_type=jnp.float32) + b1_ref[...]
    y = jnp.maximum(y, 0.0)
    y = jnp.dot(y, w2_ref[...], preferred_element_type=jnp.float32) + b2_ref[...]
    y = jnp.maximum(y, 0.0)
    out_ref[...] = y
    s = jnp.sum(y, axis=0, keepdims=True)
    q = jnp.sum(y * y, axis=0, keepdims=True)

    @pl.when(i == 0)
    def _():
        s_ref[...] = s
        q_ref[...] = q

    @pl.when(i > 0)
    def _():
        s_ref[...] += s
        q_ref[...] += q


def _mlp_hom_body(a0, a1, di_ref, w0, b0, w1, b1, w2, b2,
                  out_ref, s_ref, q_ref):
    di = di_ref[...]
    xq = [a[...] * di for a in (a0, a1)]
    _mlp_tail(xq, w0, b0, w1, b1, w2, b2, out_ref, s_ref, q_ref)


def _mlp_het_body(f0, f1, a0, a1, di_ref, w0, b0, w1, b1,
                  w2, b2, out_ref, s_ref, q_ref):
    di = di_ref[...]
    xq = [f[...] - a[...] * di for f, a in zip((f0, f1), (a0, a1))]
    _mlp_tail(xq, w0, b0, w1, b1, w2, b2, out_ref, s_ref, q_ref)


def _mlp_single_body(h0, h1, w0, b0, w1, b1, w2, b2,
                     out_ref, s_ref, q_ref):
    _mlp_tail([h0[...], h1[...]],
              w0, b0, w1, b1, w2, b2, out_ref, s_ref, q_ref)


def _mlp_call(body, row_args, row_specs, dinv_col, p, blk):
    n = row_args[0].shape[0]
    h = p["W1"].shape[0]
    args = list(row_args)
    specs = list(row_specs)
    if dinv_col is not None:
        args.append(dinv_col)
        specs.append(pl.BlockSpec((blk, 1), lambda i: (i, 0)))
    d0 = p["W0"].shape[0]
    args += [p["W0"], p["b0"].reshape(1, h), p["W1"], p["b1"].reshape(1, h),
             p["W2"], p["b2"].reshape(1, h)]
    specs += [pl.BlockSpec((d0, h), lambda i: (0, 0)),
              pl.BlockSpec((1, h), lambda i: (0, 0)),
              pl.BlockSpec((h, h), lambda i: (0, 0)),
              pl.BlockSpec((1, h), lambda i: (0, 0)),
              pl.BlockSpec((h, h), lambda i: (0, 0)),
              pl.BlockSpec((1, h), lambda i: (0, 0))]
    return pl.pallas_call(
        body,
        grid=(n // blk,),
        in_specs=specs,
        out_specs=[pl.BlockSpec((blk, h), lambda i: (i, 0)),
                   pl.BlockSpec((1, h), lambda i: (0, 0)),
                   pl.BlockSpec((1, h), lambda i: (0, 0))],
        out_shape=[jax.ShapeDtypeStruct((n, h), jnp.float32),
                   jax.ShapeDtypeStruct((1, h), jnp.float32),
                   jax.ShapeDtypeStruct((1, h), jnp.float32)],
    )(*args)


def _combine_body(lh_ref, le_ref, ls_ref, sh_ref, qh_ref, se_ref, qe_ref,
                  ss_ref, qs_ref, gh_ref, bh_ref, ge_ref, be_ref, gs_ref,
                  bs_ref, f_ref, gw_ref, gb_ref, fw_ref, fb_ref, t_ref,
                  lab_ref, cnt_ref, logit_ref, kl_ref, ce_ref, s0_ref,
                  s1_ref, s2_ref, loss_ref, *, n, nt, ngrid):
    i = pl.program_id(0)

    def norm(l_ref, s_ref, q_ref, g_ref, b_ref):
        mu = s_ref[...] / n
        var = q_ref[...] / n - mu * mu
        sc = g_ref[...] * lax.rsqrt(var + EPS)
        return l_ref[...] * sc + (b_ref[...] - mu * sc)

    xh = norm(lh_ref, sh_ref, qh_ref, gh_ref, bh_ref)
    xe = norm(le_ref, se_ref, qe_ref, ge_ref, be_ref)
    xs = norm(ls_ref, ss_ref, qs_ref, gs_ref, bs_ref)

    z = (jnp.dot(f_ref[...], gw_ref[...], preferred_element_type=jnp.float32)
         + gb_ref[...])
    zm = jnp.max(z, axis=1, keepdims=True)
    ez = jnp.exp(z - zm)
    cf = ez / jnp.sum(ez, axis=1, keepdims=True)
    c0 = cf[:, 0:1]
    c1 = cf[:, 1:2]
    c2 = cf[:, 2:3]
    m = jnp.minimum(c0, jnp.minimum(c1, c2))
    e2 = c2 <= m
    e1 = jnp.logical_and(jnp.logical_not(e2), c1 <= m)
    e0 = jnp.logical_and(jnp.logical_not(e2), jnp.logical_not(e1))
    mk0 = jnp.where(e0, c0 * -100000.0, c0)
    mk1 = jnp.where(e1, c1 * -100000.0, c1)
    mk2 = jnp.where(e2, c2 * -100000.0, c2)
    mx = jnp.maximum(mk0, jnp.maximum(mk1, mk2))
    x0 = jnp.exp(mk0 - mx)
    x1 = jnp.exp(mk1 - mx)
    x2 = jnp.exp(mk2 - mx)
    zs = x0 + x1 + x2
    w0 = x0 / zs
    w1 = x1 / zs
    w2 = x2 / zs

    emb = w0 * xh + w1 * xe + w2 * xs
    lr = (jnp.dot(emb, fw_ref[...], preferred_element_type=jnp.float32)
          + fb_ref[...])
    lm = jnp.max(lr, axis=1, keepdims=True)
    el = jnp.exp(lr - lm)
    sm = el / jnp.sum(el, axis=1, keepdims=True)
    logit_ref[...] = sm

    t = sm / TAU
    kl = jnp.sum(t * (jnp.log(t) - t_ref[...] / TAU)).reshape(1, 1)

    mx2 = jnp.max(sm, axis=1, keepdims=True)
    logp = sm - mx2 - jnp.log(jnp.sum(jnp.exp(sm - mx2), axis=1, keepdims=True))
    lanes = lax.broadcasted_iota(jnp.int32, sm.shape, 1)
    pick = jnp.sum(jnp.where(lanes == lab_ref[...], logp, 0.0), axis=1,
                   keepdims=True)
    ce = (-jnp.sum(cnt_ref[...] * pick)).reshape(1, 1)

    s0 = jnp.sum(w0).reshape(1, 1)
    s1 = jnp.sum(w1).reshape(1, 1)
    s2 = jnp.sum(w2).reshape(1, 1)

    @pl.when(i == 0)
    def _():
        kl_ref[...] = kl
        ce_ref[...] = ce
        s0_ref[...] = s0
        s1_ref[...] = s1
        s2_ref[...] = s2

    @pl.when(i > 0)
    def _():
        kl_ref[...] += kl
        ce_ref[...] += ce
        s0_ref[...] += s0
        s1_ref[...] += s1
        s2_ref[...] += s2

    @pl.when(i == ngrid - 1)
    def _():
        loss1 = kl_ref[...] / n
        loss2 = ce_ref[...] / nt
        third = 1.0 / 3.0
        aux = (jnp.abs(s0_ref[...] / n - third)
               + jnp.abs(s1_ref[...] / n - third)
               + jnp.abs(s2_ref[...] / n - third))
        loss_ref[...] = (LAMBDA1 * loss1 + (1.0 - LAMBDA1) * loss2
                         + LAMBDA2 * aux)


def _combine(lh, le, ls, stats, bns, feature, gwp, gbp, fw, fb, teacher,
             lab_col, cnt_col, nt, blk):
    n, h = lh.shape
    c = fw.shape[1]
    d = feature.shape[1]
    ngrid = n // blk
    sh, qh, se, qe, ss, qs = stats
    gh, bh, ge, be, gs, bs = bns
    row = lambda w: pl.BlockSpec((blk, w), lambda i: (i, 0))
    one = lambda w: pl.BlockSpec((1, w), lambda i: (0, 0))
    scl = lambda: pl.BlockSpec((1, 1), lambda i: (0, 0))
    return pl.pallas_call(
        functools.partial(_combine_body, n=float(n), nt=float(nt),
                          ngrid=ngrid),
        grid=(ngrid,),
        in_specs=[row(h), row(h), row(h),
                  one(h), one(h), one(h), one(h), one(h), one(h),
                  one(h), one(h), one(h), one(h), one(h), one(h),
                  row(d), pl.BlockSpec((d, 128), lambda i: (0, 0)), one(128),
                  pl.BlockSpec((h, c), lambda i: (0, 0)), one(c),
                  row(c), row(1), row(1)],
        out_specs=[row(c), scl(), scl(), scl(), scl(), scl(), scl()],
        out_shape=[jax.ShapeDtypeStruct((n, c), jnp.float32)]
                  + [jax.ShapeDtypeStruct((1, 1), jnp.float32)] * 6,
    )(lh, le, ls, sh, qh, se, qe, ss, qs, gh, bh, ge, be, gs, bs,
      feature, gwp, gbp, fw, fb, teacher, lab_col, cnt_col)


def kernel(feature, edge_index, label, train_nodes, teacher_logit, params):
    n, d = feature.shape
    e = edge_index.shape[1]
    nt = train_nodes.shape[0]
    h = params["fin_W"].shape[0]
    c = params["fin_W"].shape[1]
    dq = d // NQ

    fsum, fsq = _colstats(feature, 2000)
    hq = _bnapply(feature, fsum, fsq,
                  params["bn_in_g"].reshape(1, d),
                  params["bn_in_b"].reshape(1, d), 2000)

    KCH = 4096
    ep = -(-e // KCH) * KCH
    src_p = jnp.concatenate(
        [edge_index[0], jnp.zeros((ep - e,), jnp.int32)])
    dst_p = jnp.concatenate(
        [edge_index[1], jnp.full((ep - e,), n, jnp.int32)])
    ntp = -(-nt // 128) * 128
    tn_p = jnp.concatenate([train_nodes, jnp.full((ntp - nt,), n, jnp.int32)])

    sc_agg = _sc_agg_build(n, ep, dq)
    agg0, agg1 = sc_agg(hq[0], hq[1], src_p, dst_p)
    aggq = [agg0, agg1]
    dinv_col = _hist(dst_p, 4096, True).reshape(16384, 1)[:n]
    cnt_col = _hist(tn_p, ntp, False).reshape(16384, 1)[:n]

    MB = 1000
    aspec = [pl.BlockSpec((MB, dq), lambda i: (i, 0)) for _ in range(NQ)]
    fq = [feature[:, j * dq:(j + 1) * dq] for j in range(NQ)]

    lh, sh_, qh_ = _mlp_call(_mlp_hom_body, aggq, aspec, dinv_col,
                             params["hom"], MB)
    le, se_, qe_ = _mlp_call(_mlp_het_body, fq + aggq,
                             aspec + aspec, dinv_col, params["het"], MB)
    ls, ss_, qs_ = _mlp_call(_mlp_single_body, list(hq), aspec, None,
                             params["single"], MB)

    gwp = jnp.zeros((d, 128), jnp.float32).at[:, :3].set(params["gate_W"])
    gbp = jnp.full((1, 128), -1e30, jnp.float32).at[0, :3].set(params["gate_b"])
    stats = (sh_, qh_, se_, qe_, ss_, qs_)
    bns = (params["hom"]["bn_g"].reshape(1, h), params["hom"]["bn_b"].reshape(1, h),
           params["het"]["bn_g"].reshape(1, h), params["het"]["bn_b"].reshape(1, h),
           params["single"]["bn_g"].reshape(1, h), params["single"]["bn_b"].reshape(1, h))
    logit, _kl, _ce, _s0, _s1, _s2, loss = _combine(
        lh, le, ls, stats, bns, feature, gwp, gbp,
        params["fin_W"], params["fin_b"].reshape(1, c), teacher_logit,
        label.reshape(n, 1), cnt_col, nt, 400)
    return logit, loss.reshape(())


# speedup vs baseline: 2.7529x; 1.1078x over previous
import functools

import jax
import jax.numpy as jnp
from jax import lax
from jax.experimental import pallas as pl
from jax.experimental.pallas import tpu as pltpu
from jax.experimental.pallas import tpu_sc as plsc

TAU = 1.0
LAMBDA1 = 0.5
LAMBDA2 = 0.3
EPS = 1e-5

NC = 2
NS = 16
NQ = 2


def _colstats_body(x_ref, s_ref, q_ref):
    i = pl.program_id(0)
    x = x_ref[...]
    s = jnp.sum(x, axis=0, keepdims=True)
    q = jnp.sum(x * x, axis=0, keepdims=True)

    @pl.when(i == 0)
    def _():
        s_ref[...] = s
        q_ref[...] = q

    @pl.when(i > 0)
    def _():
        s_ref[...] += s
        q_ref[...] += q


def _colstats(x, blk):
    n, d = x.shape
    return pl.pallas_call(
        _colstats_body,
        grid=(n // blk,),
        in_specs=[pl.BlockSpec((blk, d), lambda i: (i, 0))],
        out_specs=[pl.BlockSpec((1, d), lambda i: (0, 0)),
                   pl.BlockSpec((1, d), lambda i: (0, 0))],
        out_shape=[jax.ShapeDtypeStruct((1, d), jnp.float32),
                   jax.ShapeDtypeStruct((1, d), jnp.float32)],
    )(x)


def _bnapply_body(x_ref, s_ref, q_ref, g_ref, b_ref, *h_refs, n):
    mu = s_ref[...] / n
    var = q_ref[...] / n - mu * mu
    sc = g_ref[...] * lax.rsqrt(var + EPS)
    sh = b_ref[...] - mu * sc
    h = x_ref[...] * sc + sh
    dq = h.shape[1] // NQ
    for j, h_ref in enumerate(h_refs):
        h_ref[...] = h[:, j * dq:(j + 1) * dq]


def _bnapply(x, s, q, g, b, blk):
    n, d = x.shape
    dq = d // NQ
    return pl.pallas_call(
        functools.partial(_bnapply_body, n=float(n)),
        grid=(n // blk,),
        in_specs=[pl.BlockSpec((blk, d), lambda i: (i, 0)),
                  pl.BlockSpec((1, d), lambda i: (0, 0)),
                  pl.BlockSpec((1, d), lambda i: (0, 0)),
                  pl.BlockSpec((1, d), lambda i: (0, 0)),
                  pl.BlockSpec((1, d), lambda i: (0, 0))],
        out_specs=[pl.BlockSpec((blk, dq), lambda i: (i, 0))] * NQ,
        out_shape=[jax.ShapeDtypeStruct((n, dq), jnp.float32)] * NQ,
    )(x, s, q, g, b)


def _sc_agg_build(n, e, dh):
    K = 64
    EW = e // NS
    NCH = EW // K
    ZBR = 8
    NZ = n // ZBR
    ZPER = -(-NZ // NS)
    mesh = plsc.VectorSubcoreMesh(core_axis_name="c", subcore_axis_name="s")

    @functools.partial(
        pl.kernel,
        mesh=mesh,
        out_type=[jax.ShapeDtypeStruct((n, dh), jnp.float32)] * NQ,
        scratch_types=[
            pltpu.VMEM((K,), jnp.int32),
            pltpu.VMEM((K,), jnp.int32),
            pltpu.VMEM((K, dh), jnp.float32),
            pltpu.VMEM((K,), jnp.int32),
            pltpu.VMEM((K,), jnp.int32),
            pltpu.VMEM((K, dh), jnp.float32),
            pltpu.VMEM((ZBR, dh), jnp.float32),
            pltpu.VMEM((ZBR, dh), jnp.float32),
            pltpu.VMEM_SHARED((n + 8, dh), jnp.float32),
            pltpu.SemaphoreType.DMA,
            pltpu.SemaphoreType.DMA,
        ],
    )
    def sc_agg(h0, h1, src, dst, agg0, agg1,
               src_v0, dst_v0, rows_v0, src_v1, dst_v1, rows_v1,
               zero_v, bounce_v, acc, sem0, sem1):
        cid = lax.axis_index("c")
        sid = lax.axis_index("s")
        z16f = jnp.zeros((16,), jnp.float32)

        def zrow(i, _):
            r = i // (dh // 16)
            c16 = i % (dh // 16)
            zero_v[r, pl.ds(c16 * 16, 16)] = z16f
            return _
        lax.fori_loop(0, ZBR * (dh // 16), zrow, None)

        def zacc(t, _):
            j = sid + t * NS

            @pl.when(j < NZ)
            def _():
                pltpu.sync_copy(zero_v, acc.at[pl.ds(j * ZBR, ZBR)])
            return _
        lax.fori_loop(0, ZPER, zacc, None)
        plsc.subcore_barrier()

        def scan_edges(h_ref):
            def prime(c, sv, dv, rv, sem):
                base = sid * EW + c * K
                pltpu.sync_copy(src.at[pl.ds(base, K)], sv)
                pltpu.sync_copy(dst.at[pl.ds(base, K)], dv)
                pltpu.async_copy(h_ref.at[sv], rv, sem)

            prime(0, src_v0, dst_v0, rows_v0, sem0)

            def pair(c2, _):
                c0 = c2 * 2

                prime(c0 + 1, src_v1, dst_v1, rows_v1, sem1)
                pltpu.make_async_copy(h_ref.at[src_v0], rows_v0, sem0).wait()
                pltpu.sync_copy(rows_v0, acc.at[dst_v0], add=True)

                @pl.when(c0 + 2 < NCH)
                def _():
                    prime(c0 + 2, src_v0, dst_v0, rows_v0, sem0)
                pltpu.make_async_copy(h_ref.at[src_v1], rows_v1, sem1).wait()
                pltpu.sync_copy(rows_v1, acc.at[dst_v1], add=True)
                return _
            lax.fori_loop(0, NCH // 2, pair, None)

        @pl.when(cid == 0)
        def _():
            scan_edges(h0)

        @pl.when(cid == 1)
        def _():
            scan_edges(h1)

        plsc.subcore_barrier()

        def flush(agg_ref):
            def go(t, _):
                j = sid + t * NS

                @pl.when(j < NZ)
                def _():
                    pltpu.sync_copy(acc.at[pl.ds(j * ZBR, ZBR)], bounce_v)
                    pltpu.sync_copy(bounce_v, agg_ref.at[pl.ds(j * ZBR, ZBR)])
                return _
            lax.fori_loop(0, ZPER, go, None)

        @pl.when(cid == 0)
        def _():
            flush(agg0)

        @pl.when(cid == 1)
        def _():
            flush(agg1)

    return sc_agg


def _hist_body(d_ref, out_ref, *, ngrid, recip):
    i = pl.program_id(0)
    d = d_ref[0]
    cols = lax.broadcasted_iota(jnp.int32, (d.shape[0], 128), 1)
    hi = lax.shift_right_logical(d, 7)
    lo = jnp.bitwise_and(d, 127)
    a = (hi == cols).astype(jnp.float32)
    b = (lo == cols).astype(jnp.float32)
    m = lax.dot_general(a, b, (((0,), (0,)), ((), ())),
                        preferred_element_type=jnp.float32)

    @pl.when(i == 0)
    def _():
        out_ref[...] = m

    @pl.when(i > 0)
    def _():
        out_ref[...] += m

    if recip:
        @pl.when(i == ngrid - 1)
        def _():
            out_ref[...] = 1.0 / jnp.maximum(out_ref[...], 1.0)


def _hist(idx, blk, recip):
    (e,) = idx.shape
    ngrid = e // blk
    idx3 = idx.reshape(ngrid, blk, 1)
    return pl.pallas_call(
        functools.partial(_hist_body, ngrid=ngrid, recip=recip),
        grid=(ngrid,),
        in_specs=[pl.BlockSpec((1, blk, 1), lambda i: (i, 0, 0))],
        out_specs=pl.BlockSpec((128, 128), lambda i: (0, 0)),
        out_shape=jax.ShapeDtypeStruct((128, 128), jnp.float32),
    )(idx3)


def _mlp_tail(xq, w0_ref, b0_ref, w1_ref, b1_ref, w2_ref, b2_ref,
              out_ref, s_ref, q_ref):
    i = pl.program_id(0)
    w0 = w0_ref[...]
    dq = xq[0].shape[1]
    y = b0_ref[...].astype(jnp.float32)
    for j, x in enumerate(xq):
        y = y + jnp.dot(x, w0[j * dq:(j + 1) * dq, :],
                        preferred_element_type=jnp.float32)
    y = jnp.maximum(y, 0.0)
    y = jnp.dot(y, w1_ref[...], preferred_element_type=jnp.float32) + b1_ref[...]
    y = jnp.maximum(y, 0.0)
    y = jnp.dot(y, w2_ref[...], preferred_element_type=jnp.float32) + b2_ref[...]
    y = jnp.maximum(y, 0.0)
    out_ref[...] = y
    s = jnp.sum(y, axis=0, keepdims=True)
    q = jnp.sum(y * y, axis=0, keepdims=True)

    @pl.when(i == 0)
    def _():
        s_ref[...] = s
        q_ref[...] = q

    @pl.when(i > 0)
    def _():
        s_ref[...] += s
        q_ref[...] += q


def _mlp_hom_body(a0, a1, di_ref, w0, b0, w1, b1, w2, b2,
                  out_ref, s_ref, q_ref):
    di = di_ref[...]
    xq = [a[...] * di for a in (a0, a1)]
    _mlp_tail(xq, w0, b0, w1, b1, w2, b2, out_ref, s_ref, q_ref)


def _mlp_het_body(f0, f1, a0, a1, di_ref, w0, b0, w1, b1,
                  w2, b2, out_ref, s_ref, q_ref):
    di = di_ref[...]
    xq = [f[...] - a[...] * di for f, a in zip((f0, f1), (a0, a1))]
    _mlp_tail(xq, w0, b0, w1, b1, w2, b2, out_ref, s_ref, q_ref)


def _mlp_single_body(h0, h1, w0, b0, w1, b1, w2, b2,
                     out_ref, s_ref, q_ref):
    _mlp_tail([h0[...], h1[...]],
              w0, b0, w1, b1, w2, b2, out_ref, s_ref, q_ref)


def _mlp_call(body, row_args, row_specs, dinv_col, p, blk):
    n = row_args[0].shape[0]
    h = p["W1"].shape[0]
    args = list(row_args)
    specs = list(row_specs)
    if dinv_col is not None:
        args.append(dinv_col)
        specs.append(pl.BlockSpec((blk, 1), lambda i: (i, 0)))
    d0 = p["W0"].shape[0]
    args += [p["W0"], p["b0"].reshape(1, h), p["W1"], p["b1"].reshape(1, h),
             p["W2"], p["b2"].reshape(1, h)]
    specs += [pl.BlockSpec((d0, h), lambda i: (0, 0)),
              pl.BlockSpec((1, h), lambda i: (0, 0)),
              pl.BlockSpec((h, h), lambda i: (0, 0)),
              pl.BlockSpec((1, h), lambda i: (0, 0)),
              pl.BlockSpec((h, h), lambda i: (0, 0)),
              pl.BlockSpec((1, h), lambda i: (0, 0))]
    return pl.pallas_call(
        body,
        grid=(n // blk,),
        in_specs=specs,
        out_specs=[pl.BlockSpec((blk, h), lambda i: (i, 0)),
                   pl.BlockSpec((1, h), lambda i: (0, 0)),
                   pl.BlockSpec((1, h), lambda i: (0, 0))],
        out_shape=[jax.ShapeDtypeStruct((n, h), jnp.float32),
                   jax.ShapeDtypeStruct((1, h), jnp.float32),
                   jax.ShapeDtypeStruct((1, h), jnp.float32)],
    )(*args)


def _combine_body(lh_ref, le_ref, ls_ref, sh_ref, qh_ref, se_ref, qe_ref,
                  ss_ref, qs_ref, gh_ref, bh_ref, ge_ref, be_ref, gs_ref,
                  bs_ref, f_ref, gw_ref, gb_ref, fw_ref, fb_ref, t_ref,
                  lab_ref, cnt_ref, logit_ref, kl_ref, ce_ref, s0_ref,
                  s1_ref, s2_ref, loss_ref, *, n, nt, ngrid):
    i = pl.program_id(0)

    def norm(l_ref, s_ref, q_ref, g_ref, b_ref):
        mu = s_ref[...] / n
        var = q_ref[...] / n - mu * mu
        sc = g_ref[...] * lax.rsqrt(var + EPS)
        return l_ref[...] * sc + (b_ref[...] - mu * sc)

    xh = norm(lh_ref, sh_ref, qh_ref, gh_ref, bh_ref)
    xe = norm(le_ref, se_ref, qe_ref, ge_ref, be_ref)
    xs = norm(ls_ref, ss_ref, qs_ref, gs_ref, bs_ref)

    z = (jnp.dot(f_ref[...], gw_ref[...], preferred_element_type=jnp.float32)
         + gb_ref[...])
    zm = jnp.max(z, axis=1, keepdims=True)
    ez = jnp.exp(z - zm)
    cf = ez / jnp.sum(ez, axis=1, keepdims=True)
    c0 = cf[:, 0:1]
    c1 = cf[:, 1:2]
    c2 = cf[:, 2:3]
    m = jnp.minimum(c0, jnp.minimum(c1, c2))
    e2 = c2 <= m
    e1 = jnp.logical_and(jnp.logical_not(e2), c1 <= m)
    e0 = jnp.logical_and(jnp.logical_not(e2), jnp.logical_not(e1))
    mk0 = jnp.where(e0, c0 * -100000.0, c0)
    mk1 = jnp.where(e1, c1 * -100000.0, c1)
    mk2 = jnp.where(e2, c2 * -100000.0, c2)
    mx = jnp.maximum(mk0, jnp.maximum(mk1, mk2))
    x0 = jnp.exp(mk0 - mx)
    x1 = jnp.exp(mk1 - mx)
    x2 = jnp.exp(mk2 - mx)
    zs = x0 + x1 + x2
    w0 = x0 / zs
    w1 = x1 / zs
    w2 = x2 / zs

    emb = w0 * xh + w1 * xe + w2 * xs
    lr = (jnp.dot(emb, fw_ref[...], preferred_element_type=jnp.float32)
          + fb_ref[...])
    lm = jnp.max(lr, axis=1, keepdims=True)
    el = jnp.exp(lr - lm)
    sm = el / jnp.sum(el, axis=1, keepdims=True)
    logit_ref[...] = sm

    t = sm / TAU
    kl = jnp.sum(t * (jnp.log(t) - t_ref[...] / TAU)).reshape(1, 1)

    mx2 = jnp.max(sm, axis=1, keepdims=True)
    logp = sm - mx2 - jnp.log(jnp.sum(jnp.exp(sm - mx2), axis=1, keepdims=True))
    lanes = lax.broadcasted_iota(jnp.int32, sm.shape, 1)
    pick = jnp.sum(jnp.where(lanes == lab_ref[...], logp, 0.0), axis=1,
                   keepdims=True)
    ce = (-jnp.sum(cnt_ref[...] * pick)).reshape(1, 1)

    s0 = jnp.sum(w0).reshape(1, 1)
    s1 = jnp.sum(w1).reshape(1, 1)
    s2 = jnp.sum(w2).reshape(1, 1)

    @pl.when(i == 0)
    def _():
        kl_ref[...] = kl
        ce_ref[...] = ce
        s0_ref[...] = s0
        s1_ref[...] = s1
        s2_ref[...] = s2

    @pl.when(i > 0)
    def _():
        kl_ref[...] += kl
        ce_ref[...] += ce
        s0_ref[...] += s0
        s1_ref[...] += s1
        s2_ref[...] += s2

    @pl.when(i == ngrid - 1)
    def _():
        loss1 = kl_ref[...] / n
        loss2 = ce_ref[...] / nt
        third = 1.0 / 3.0
        aux = (jnp.abs(s0_ref[...] / n - third)
               + jnp.abs(s1_ref[...] / n - third)
               + jnp.abs(s2_ref[...] / n - third))
        loss_ref[...] = (LAMBDA1 * loss1 + (1.0 - LAMBDA1) * loss2
                         + LAMBDA2 * aux)


def _combine(lh, le, ls, stats, bns, feature, gwp, gbp, fw, fb, teacher,
             lab_col, cnt_col, nt, blk):
    n, h = lh.shape
    c = fw.shape[1]
    d = feature.shape[1]
    ngrid = n // blk
    sh, qh, se, qe, ss, qs = stats
    gh, bh, ge, be, gs, bs = bns
    row = lambda w: pl.BlockSpec((blk, w), lambda i: (i, 0))
    one = lambda w: pl.BlockSpec((1, w), lambda i: (0, 0))
    scl = lambda: pl.BlockSpec((1, 1), lambda i: (0, 0))
    return pl.pallas_call(
        functools.partial(_combine_body, n=float(n), nt=float(nt),
                          ngrid=ngrid),
        grid=(ngrid,),
        in_specs=[row(h), row(h), row(h),
                  one(h), one(h), one(h), one(h), one(h), one(h),
                  one(h), one(h), one(h), one(h), one(h), one(h),
                  row(d), pl.BlockSpec((d, 128), lambda i: (0, 0)), one(128),
                  pl.BlockSpec((h, c), lambda i: (0, 0)), one(c),
                  row(c), row(1), row(1)],
        out_specs=[row(c), scl(), scl(), scl(), scl(), scl(), scl()],
        out_shape=[jax.ShapeDtypeStruct((n, c), jnp.float32)]
                  + [jax.ShapeDtypeStruct((1, 1), jnp.float32)] * 6,
    )(lh, le, ls, sh, qh, se, qe, ss, qs, gh, bh, ge, be, gs, bs,
      feature, gwp, gbp, fw, fb, teacher, lab_col, cnt_col)


def kernel(feature, edge_index, label, train_nodes, teacher_logit, params):
    n, d = feature.shape
    e = edge_index.shape[1]
    nt = train_nodes.shape[0]
    h = params["fin_W"].shape[0]
    c = params["fin_W"].shape[1]
    dq = d // NQ

    fsum, fsq = _colstats(feature, 2000)
    hq = _bnapply(feature, fsum, fsq,
                  params["bn_in_g"].reshape(1, d),
                  params["bn_in_b"].reshape(1, d), 2000)

    KCH = 4096
    ep = -(-e // KCH) * KCH
    src_p = jnp.concatenate(
        [edge_index[0], jnp.zeros((ep - e,), jnp.int32)])
    dst_p = jnp.concatenate(
        [edge_index[1], jnp.full((ep - e,), n, jnp.int32)])
    ntp = -(-nt // 128) * 128
    tn_p = jnp.concatenate([train_nodes, jnp.full((ntp - nt,), n, jnp.int32)])

    sc_agg = _sc_agg_build(n, ep, dq)
    agg0, agg1 = sc_agg(hq[0], hq[1], src_p, dst_p)
    aggq = [agg0, agg1]
    dinv_col = _hist(dst_p, 4096, True).reshape(16384, 1)[:n]
    cnt_col = _hist(tn_p, ntp, False).reshape(16384, 1)[:n]

    MB = 1000
    aspec = [pl.BlockSpec((MB, dq), lambda i: (i, 0)) for _ in range(NQ)]
    fq = [feature[:, j * dq:(j + 1) * dq] for j in range(NQ)]

    lh, sh_, qh_ = _mlp_call(_mlp_hom_body, aggq, aspec, dinv_col,
                             params["hom"], MB)
    le, se_, qe_ = _mlp_call(_mlp_het_body, fq + aggq,
                             aspec + aspec, dinv_col, params["het"], MB)
    ls, ss_, qs_ = _mlp_call(_mlp_single_body, list(hq), aspec, None,
                             params["single"], MB)

    gwp = jnp.zeros((d, 128), jnp.float32).at[:, :3].set(params["gate_W"])
    gbp = jnp.full((1, 128), -1e30, jnp.float32).at[0, :3].set(params["gate_b"])
    stats = (sh_, qh_, se_, qe_, ss_, qs_)
    bns = (params["hom"]["bn_g"].reshape(1, h), params["hom"]["bn_b"].reshape(1, h),
           params["het"]["bn_g"].reshape(1, h), params["het"]["bn_b"].reshape(1, h),
           params["single"]["bn_g"].reshape(1, h), params["single"]["bn_b"].reshape(1, h))
    logit, _kl, _ce, _s0, _s1, _s2, loss = _combine(
        lh, le, ls, stats, bns, feature, gwp, gbp,
        params["fin_W"], params["fin_b"].reshape(1, c), teacher_logit,
        label.reshape(n, 1), cnt_col, nt, 400)
    return logit, loss.reshape(())


# speedup vs baseline: 2.8516x; 1.0358x over previous
import functools

import jax
import jax.numpy as jnp
from jax import lax
from jax.experimental import pallas as pl
from jax.experimental.pallas import tpu as pltpu
from jax.experimental.pallas import tpu_sc as plsc

TAU = 1.0
LAMBDA1 = 0.5
LAMBDA2 = 0.3
EPS = 1e-5

NC = 2
NS = 16
NQ = 2


def _colstats_body(x_ref, s_ref, q_ref):
    i = pl.program_id(0)
    x = x_ref[...]
    s = jnp.sum(x, axis=0, keepdims=True)
    q = jnp.sum(x * x, axis=0, keepdims=True)

    @pl.when(i == 0)
    def _():
        s_ref[...] = s
        q_ref[...] = q

    @pl.when(i > 0)
    def _():
        s_ref[...] += s
        q_ref[...] += q


def _colstats(x, blk):
    n, d = x.shape
    return pl.pallas_call(
        _colstats_body,
        grid=(n // blk,),
        in_specs=[pl.BlockSpec((blk, d), lambda i: (i, 0))],
        out_specs=[pl.BlockSpec((1, d), lambda i: (0, 0)),
                   pl.BlockSpec((1, d), lambda i: (0, 0))],
        out_shape=[jax.ShapeDtypeStruct((1, d), jnp.float32),
                   jax.ShapeDtypeStruct((1, d), jnp.float32)],
    )(x)


def _bnapply_body(x_ref, s_ref, q_ref, g_ref, b_ref, *h_refs, n):
    mu = s_ref[...] / n
    var = q_ref[...] / n - mu * mu
    sc = g_ref[...] * lax.rsqrt(var + EPS)
    sh = b_ref[...] - mu * sc
    h = x_ref[...] * sc + sh
    dq = h.shape[1] // NQ
    for j, h_ref in enumerate(h_refs):
        h_ref[...] = h[:, j * dq:(j + 1) * dq]


def _bnapply(x, s, q, g, b, blk):
    n, d = x.shape
    dq = d // NQ
    return pl.pallas_call(
        functools.partial(_bnapply_body, n=float(n)),
        grid=(n // blk,),
        in_specs=[pl.BlockSpec((blk, d), lambda i: (i, 0)),
                  pl.BlockSpec((1, d), lambda i: (0, 0)),
                  pl.BlockSpec((1, d), lambda i: (0, 0)),
                  pl.BlockSpec((1, d), lambda i: (0, 0)),
                  pl.BlockSpec((1, d), lambda i: (0, 0))],
        out_specs=[pl.BlockSpec((blk, dq), lambda i: (i, 0))] * NQ,
        out_shape=[jax.ShapeDtypeStruct((n, dq), jnp.float32)] * NQ,
    )(x, s, q, g, b)


def _sc_agg_build(n, e, dh):
    K = 64
    EW = e // NS
    NCH = EW // K
    ZBR = 8
    NZ = n // ZBR
    ZPER = -(-NZ // NS)
    mesh = plsc.VectorSubcoreMesh(core_axis_name="c", subcore_axis_name="s")

    @functools.partial(
        pl.kernel,
        mesh=mesh,
        out_type=[jax.ShapeDtypeStruct((n, dh), jnp.float32)] * NQ,
        scratch_types=[
            pltpu.VMEM((K,), jnp.int32),
            pltpu.VMEM((K,), jnp.int32),
            pltpu.VMEM((K, dh), jnp.float32),
            pltpu.VMEM((K,), jnp.int32),
            pltpu.VMEM((K,), jnp.int32),
            pltpu.VMEM((K, dh), jnp.float32),
            pltpu.VMEM((ZBR, dh), jnp.float32),
            pltpu.VMEM((ZBR, dh), jnp.float32),
            pltpu.VMEM_SHARED((n + 8, dh), jnp.float32),
            pltpu.SemaphoreType.DMA,
            pltpu.SemaphoreType.DMA,
        ],
    )
    def sc_agg(h0, h1, src, dst, agg0, agg1,
               src_v0, dst_v0, rows_v0, src_v1, dst_v1, rows_v1,
               zero_v, bounce_v, acc, sem0, sem1):
        cid = lax.axis_index("c")
        sid = lax.axis_index("s")
        z16f = jnp.zeros((16,), jnp.float32)

        def zrow(i, _):
            r = i // (dh // 16)
            c16 = i % (dh // 16)
            zero_v[r, pl.ds(c16 * 16, 16)] = z16f
            return _
        lax.fori_loop(0, ZBR * (dh // 16), zrow, None)

        def zacc(t, _):
            j = sid + t * NS

            @pl.when(j < NZ)
            def _():
                pltpu.sync_copy(zero_v, acc.at[pl.ds(j * ZBR, ZBR)])
            return _
        lax.fori_loop(0, ZPER, zacc, None)
        plsc.subcore_barrier()

        def scan_edges(h_ref):
            def prime(c, sv, dv, rv, sem):
                base = sid * EW + c * K
                pltpu.sync_copy(src.at[pl.ds(base, K)], sv)
                pltpu.sync_copy(dst.at[pl.ds(base, K)], dv)
                pltpu.async_copy(h_ref.at[sv], rv, sem)

            prime(0, src_v0, dst_v0, rows_v0, sem0)

            def pair(c2, _):
                c0 = c2 * 2

                prime(c0 + 1, src_v1, dst_v1, rows_v1, sem1)
                pltpu.make_async_copy(h_ref.at[src_v0], rows_v0, sem0).wait()
                pltpu.sync_copy(rows_v0, acc.at[dst_v0], add=True)

                @pl.when(c0 + 2 < NCH)
                def _():
                    prime(c0 + 2, src_v0, dst_v0, rows_v0, sem0)
                pltpu.make_async_copy(h_ref.at[src_v1], rows_v1, sem1).wait()
                pltpu.sync_copy(rows_v1, acc.at[dst_v1], add=True)
                return _
            lax.fori_loop(0, NCH // 2, pair, None)

        @pl.when(cid == 0)
        def _():
            scan_edges(h0)

        @pl.when(cid == 1)
        def _():
            scan_edges(h1)

        plsc.subcore_barrier()

        def flush(agg_ref):
            def go(t, _):
                j = sid + t * NS

                @pl.when(j < NZ)
                def _():
                    pltpu.sync_copy(acc.at[pl.ds(j * ZBR, ZBR)], bounce_v)
                    pltpu.sync_copy(bounce_v, agg_ref.at[pl.ds(j * ZBR, ZBR)])
                return _
            lax.fori_loop(0, ZPER, go, None)

        @pl.when(cid == 0)
        def _():
            flush(agg0)

        @pl.when(cid == 1)
        def _():
            flush(agg1)

    return sc_agg


def _hist_body(d_ref, out_ref, *, ngrid, recip):
    i = pl.program_id(0)
    d = d_ref[0]
    cols = lax.broadcasted_iota(jnp.int32, (d.shape[0], 128), 1)
    hi = lax.shift_right_logical(d, 7)
    lo = jnp.bitwise_and(d, 127)
    a = (hi == cols).astype(jnp.float32)
    b = (lo == cols).astype(jnp.float32)
    m = lax.dot_general(a, b, (((0,), (0,)), ((), ())),
                        preferred_element_type=jnp.float32)

    @pl.when(i == 0)
    def _():
        out_ref[...] = m

    @pl.when(i > 0)
    def _():
        out_ref[...] += m

    if recip:
        @pl.when(i == ngrid - 1)
        def _():
            out_ref[...] = 1.0 / jnp.maximum(out_ref[...], 1.0)


def _hist(idx, blk, recip):
    (e,) = idx.shape
    ngrid = e // blk
    idx3 = idx.reshape(ngrid, blk, 1)
    return pl.pallas_call(
        functools.partial(_hist_body, ngrid=ngrid, recip=recip),
        grid=(ngrid,),
        in_specs=[pl.BlockSpec((1, blk, 1), lambda i: (i, 0, 0))],
        out_specs=pl.BlockSpec((128, 128), lambda i: (0, 0)),
        out_shape=jax.ShapeDtypeStruct((128, 128), jnp.float32),
    )(idx3)


def _mlp_tail(xq, w0_ref, b0_ref, w1_ref, b1_ref, w2_ref, b2_ref,
              out_ref, s_ref, q_ref):
    i = pl.program_id(0)
    w0 = w0_ref[...]
    dq = xq[0].shape[1]
    y = b0_ref[...].astype(jnp.float32)
    for j, x in enumerate(xq):
        y = y + jnp.dot(x.astype(jnp.bfloat16), w0[j * dq:(j + 1) * dq, :],
                        preferred_element_type=jnp.float32)
    y = jnp.maximum(y, 0.0)
    y = jnp.dot(y.astype(jnp.bfloat16), w1_ref[...],
                preferred_element_type=jnp.float32) + b1_ref[...]
    y = jnp.maximum(y, 0.0)
    y = jnp.dot(y.astype(jnp.bfloat16), w2_ref[...],
                preferred_element_type=jnp.float32) + b2_ref[...]
    y = jnp.maximum(y, 0.0)
    out_ref[...] = y
    s = jnp.sum(y, axis=0, keepdims=True)
    q = jnp.sum(y * y, axis=0, keepdims=True)

    @pl.when(i == 0)
    def _():
        s_ref[...] = s
        q_ref[...] = q

    @pl.when(i > 0)
    def _():
        s_ref[...] += s
        q_ref[...] += q


def _mlp_hom_body(a0, a1, di_ref, w0, b0, w1, b1, w2, b2,
                  out_ref, s_ref, q_ref):
    di = di_ref[...]
    xq = [a[...] * di for a in (a0, a1)]
    _mlp_tail(xq, w0, b0, w1, b1, w2, b2, out_ref, s_ref, q_ref)


def _mlp_het_body(f0, f1, a0, a1, di_ref, w0, b0, w1, b1,
                  w2, b2, out_ref, s_ref, q_ref):
    di = di_ref[...]
    xq = [f[...] - a[...] * di for f, a in zip((f0, f1), (a0, a1))]
    _mlp_tail(xq, w0, b0, w1, b1, w2, b2, out_ref, s_ref, q_ref)


def _mlp_single_body(h0, h1, w0, b0, w1, b1, w2, b2,
                     out_ref, s_ref, q_ref):
    _mlp_tail([h0[...], h1[...]],
              w0, b0, w1, b1, w2, b2, out_ref, s_ref, q_ref)


def _mlp_call(body, row_args, row_specs, dinv_col, p, blk):
    n = row_args[0].shape[0]
    h = p["W1"].shape[0]
    args = list(row_args)
    specs = list(row_specs)
    if dinv_col is not None:
        args.append(dinv_col)
        specs.append(pl.BlockSpec((blk, 1), lambda i: (i, 0)))
    d0 = p["W0"].shape[0]
    bf = jnp.bfloat16
    args += [p["W0"].astype(bf), p["b0"].reshape(1, h),
             p["W1"].astype(bf), p["b1"].reshape(1, h),
             p["W2"].astype(bf), p["b2"].reshape(1, h)]
    specs += [pl.BlockSpec((d0, h), lambda i: (0, 0)),
              pl.BlockSpec((1, h), lambda i: (0, 0)),
              pl.BlockSpec((h, h), lambda i: (0, 0)),
              pl.BlockSpec((1, h), lambda i: (0, 0)),
              pl.BlockSpec((h, h), lambda i: (0, 0)),
              pl.BlockSpec((1, h), lambda i: (0, 0))]
    return pl.pallas_call(
        body,
        grid=(n // blk,),
        in_specs=specs,
        out_specs=[pl.BlockSpec((blk, h), lambda i: (i, 0)),
                   pl.BlockSpec((1, h), lambda i: (0, 0)),
                   pl.BlockSpec((1, h), lambda i: (0, 0))],
        out_shape=[jax.ShapeDtypeStruct((n, h), jnp.float32),
                   jax.ShapeDtypeStruct((1, h), jnp.float32),
                   jax.ShapeDtypeStruct((1, h), jnp.float32)],
    )(*args)


def _combine_body(lh_ref, le_ref, ls_ref, sh_ref, qh_ref, se_ref, qe_ref,
                  ss_ref, qs_ref, gh_ref, bh_ref, ge_ref, be_ref, gs_ref,
                  bs_ref, f_ref, gw_ref, gb_ref, fw_ref, fb_ref, t_ref,
                  lab_ref, cnt_ref, logit_ref, kl_ref, ce_ref, s0_ref,
                  s1_ref, s2_ref, loss_ref, *, n, nt, ngrid):
    i = pl.program_id(0)

    def norm(l_ref, s_ref, q_ref, g_ref, b_ref):
        mu = s_ref[...] / n
        var = q_ref[...] / n - mu * mu
        sc = g_ref[...] * lax.rsqrt(var + EPS)
        return l_ref[...] * sc + (b_ref[...] - mu * sc)

    xh = norm(lh_ref, sh_ref, qh_ref, gh_ref, bh_ref)
    xe = norm(le_ref, se_ref, qe_ref, ge_ref, be_ref)
    xs = norm(ls_ref, ss_ref, qs_ref, gs_ref, bs_ref)

    z = (jnp.dot(f_ref[...], gw_ref[...], preferred_element_type=jnp.float32)
         + gb_ref[...])
    zm = jnp.max(z, axis=1, keepdims=True)
    ez = jnp.exp(z - zm)
    cf = ez / jnp.sum(ez, axis=1, keepdims=True)
    c0 = cf[:, 0:1]
    c1 = cf[:, 1:2]
    c2 = cf[:, 2:3]
    m = jnp.minimum(c0, jnp.minimum(c1, c2))
    e2 = c2 <= m
    e1 = jnp.logical_and(jnp.logical_not(e2), c1 <= m)
    e0 = jnp.logical_and(jnp.logical_not(e2), jnp.logical_not(e1))
    mk0 = jnp.where(e0, c0 * -100000.0, c0)
    mk1 = jnp.where(e1, c1 * -100000.0, c1)
    mk2 = jnp.where(e2, c2 * -100000.0, c2)
    mx = jnp.maximum(mk0, jnp.maximum(mk1, mk2))
    x0 = jnp.exp(mk0 - mx)
    x1 = jnp.exp(mk1 - mx)
    x2 = jnp.exp(mk2 - mx)
    zs = x0 + x1 + x2
    w0 = x0 / zs
    w1 = x1 / zs
    w2 = x2 / zs

    emb = w0 * xh + w1 * xe + w2 * xs
    lr = (jnp.dot(emb, fw_ref[...], preferred_element_type=jnp.float32)
          + fb_ref[...])
    lm = jnp.max(lr, axis=1, keepdims=True)
    el = jnp.exp(lr - lm)
    sm = el / jnp.sum(el, axis=1, keepdims=True)
    logit_ref[...] = sm

    t = sm / TAU
    kl = jnp.sum(t * (jnp.log(t) - t_ref[...] / TAU)).reshape(1, 1)

    mx2 = jnp.max(sm, axis=1, keepdims=True)
    logp = sm - mx2 - jnp.log(jnp.sum(jnp.exp(sm - mx2), axis=1, keepdims=True))
    lanes = lax.broadcasted_iota(jnp.int32, sm.shape, 1)
    pick = jnp.sum(jnp.where(lanes == lab_ref[...], logp, 0.0), axis=1,
                   keepdims=True)
    ce = (-jnp.sum(cnt_ref[...] * pick)).reshape(1, 1)

    s0 = jnp.sum(w0).reshape(1, 1)
    s1 = jnp.sum(w1).reshape(1, 1)
    s2 = jnp.sum(w2).reshape(1, 1)

    @pl.when(i == 0)
    def _():
        kl_ref[...] = kl
        ce_ref[...] = ce
        s0_ref[...] = s0
        s1_ref[...] = s1
        s2_ref[...] = s2

    @pl.when(i > 0)
    def _():
        kl_ref[...] += kl
        ce_ref[...] += ce
        s0_ref[...] += s0
        s1_ref[...] += s1
        s2_ref[...] += s2

    @pl.when(i == ngrid - 1)
    def _():
        loss1 = kl_ref[...] / n
        loss2 = ce_ref[...] / nt
        third = 1.0 / 3.0
        aux = (jnp.abs(s0_ref[...] / n - third)
               + jnp.abs(s1_ref[...] / n - third)
               + jnp.abs(s2_ref[...] / n - third))
        loss_ref[...] = (LAMBDA1 * loss1 + (1.0 - LAMBDA1) * loss2
                         + LAMBDA2 * aux)


def _combine(lh, le, ls, stats, bns, feature, gwp, gbp, fw, fb, teacher,
             lab_col, cnt_col, nt, blk):
    n, h = lh.shape
    c = fw.shape[1]
    d = feature.shape[1]
    ngrid = n // blk
    sh, qh, se, qe, ss, qs = stats
    gh, bh, ge, be, gs, bs = bns
    row = lambda w: pl.BlockSpec((blk, w), lambda i: (i, 0))
    one = lambda w: pl.BlockSpec((1, w), lambda i: (0, 0))
    scl = lambda: pl.BlockSpec((1, 1), lambda i: (0, 0))
    return pl.pallas_call(
        functools.partial(_combine_body, n=float(n), nt=float(nt),
                          ngrid=ngrid),
        grid=(ngrid,),
        in_specs=[row(h), row(h), row(h),
                  one(h), one(h), one(h), one(h), one(h), one(h),
                  one(h), one(h), one(h), one(h), one(h), one(h),
                  row(d), pl.BlockSpec((d, 128), lambda i: (0, 0)), one(128),
                  pl.BlockSpec((h, c), lambda i: (0, 0)), one(c),
                  row(c), row(1), row(1)],
        out_specs=[row(c), scl(), scl(), scl(), scl(), scl(), scl()],
        out_shape=[jax.ShapeDtypeStruct((n, c), jnp.float32)]
                  + [jax.ShapeDtypeStruct((1, 1), jnp.float32)] * 6,
    )(lh, le, ls, sh, qh, se, qe, ss, qs, gh, bh, ge, be, gs, bs,
      feature, gwp, gbp, fw, fb, teacher, lab_col, cnt_col)


def kernel(feature, edge_index, label, train_nodes, teacher_logit, params):
    n, d = feature.shape
    e = edge_index.shape[1]
    nt = train_nodes.shape[0]
    h = params["fin_W"].shape[0]
    c = params["fin_W"].shape[1]
    dq = d // NQ

    fsum, fsq = _colstats(feature, 2000)
    hq = _bnapply(feature, fsum, fsq,
                  params["bn_in_g"].reshape(1, d),
                  params["bn_in_b"].reshape(1, d), 2000)

    KCH = 4096
    ep = -(-e // KCH) * KCH
    src_p = jnp.concatenate(
        [edge_index[0], jnp.zeros((ep - e,), jnp.int32)])
    dst_p = jnp.concatenate(
        [edge_index[1], jnp.full((ep - e,), n, jnp.int32)])
    ntp = -(-nt // 128) * 128
    tn_p = jnp.concatenate([train_nodes, jnp.full((ntp - nt,), n, jnp.int32)])

    sc_agg = _sc_agg_build(n, ep, dq)
    agg0, agg1 = sc_agg(hq[0], hq[1], src_p, dst_p)
    aggq = [agg0, agg1]
    dinv_col = _hist(dst_p, 4096, True).reshape(16384, 1)[:n]
    cnt_col = _hist(tn_p, ntp, False).reshape(16384, 1)[:n]

    MB = 1000
    aspec = [pl.BlockSpec((MB, dq), lambda i: (i, 0)) for _ in range(NQ)]
    fq = [feature[:, j * dq:(j + 1) * dq] for j in range(NQ)]

    lh, sh_, qh_ = _mlp_call(_mlp_hom_body, aggq, aspec, dinv_col,
                             params["hom"], MB)
    le, se_, qe_ = _mlp_call(_mlp_het_body, fq + aggq,
                             aspec + aspec, dinv_col, params["het"], MB)
    ls, ss_, qs_ = _mlp_call(_mlp_single_body, list(hq), aspec, None,
                             params["single"], MB)

    gwp = jnp.zeros((d, 128), jnp.float32).at[:, :3].set(params["gate_W"])
    gbp = jnp.full((1, 128), -1e30, jnp.float32).at[0, :3].set(params["gate_b"])
    stats = (sh_, qh_, se_, qe_, ss_, qs_)
    bns = (params["hom"]["bn_g"].reshape(1, h), params["hom"]["bn_b"].reshape(1, h),
           params["het"]["bn_g"].reshape(1, h), params["het"]["bn_b"].reshape(1, h),
           params["single"]["bn_g"].reshape(1, h), params["single"]["bn_b"].reshape(1, h))
    logit, _kl, _ce, _s0, _s1, _s2, loss = _combine(
        lh, le, ls, stats, bns, feature, gwp, gbp,
        params["fin_W"], params["fin_b"].reshape(1, c), teacher_logit,
        label.reshape(n, 1), cnt_col, nt, 400)
    return logit, loss.reshape(())


# speedup vs baseline: 2.8987x; 1.0165x over previous
import functools

import jax
import jax.numpy as jnp
from jax import lax
from jax.experimental import pallas as pl
from jax.experimental.pallas import tpu as pltpu
from jax.experimental.pallas import tpu_sc as plsc

TAU = 1.0
LAMBDA1 = 0.5
LAMBDA2 = 0.3
EPS = 1e-5

NC = 2
NS = 16
NQ = 2


def _colstats_body(x_ref, s_ref, q_ref):
    i = pl.program_id(0)
    x = x_ref[...]
    s = jnp.sum(x, axis=0, keepdims=True)
    q = jnp.sum(x * x, axis=0, keepdims=True)

    @pl.when(i == 0)
    def _():
        s_ref[...] = s
        q_ref[...] = q

    @pl.when(i > 0)
    def _():
        s_ref[...] += s
        q_ref[...] += q


def _colstats(x, blk):
    n, d = x.shape
    return pl.pallas_call(
        _colstats_body,
        grid=(n // blk,),
        in_specs=[pl.BlockSpec((blk, d), lambda i: (i, 0))],
        out_specs=[pl.BlockSpec((1, d), lambda i: (0, 0)),
                   pl.BlockSpec((1, d), lambda i: (0, 0))],
        out_shape=[jax.ShapeDtypeStruct((1, d), jnp.float32),
                   jax.ShapeDtypeStruct((1, d), jnp.float32)],
    )(x)


def _bnapply_body(x_ref, s_ref, q_ref, g_ref, b_ref, *h_refs, n):
    mu = s_ref[...] / n
    var = q_ref[...] / n - mu * mu
    sc = g_ref[...] * lax.rsqrt(var + EPS)
    sh = b_ref[...] - mu * sc
    h = x_ref[...] * sc + sh
    dq = h.shape[1] // NQ
    for j, h_ref in enumerate(h_refs):
        h_ref[...] = h[:, j * dq:(j + 1) * dq]


def _bnapply(x, s, q, g, b, blk):
    n, d = x.shape
    dq = d // NQ
    return pl.pallas_call(
        functools.partial(_bnapply_body, n=float(n)),
        grid=(n // blk,),
        in_specs=[pl.BlockSpec((blk, d), lambda i: (i, 0)),
                  pl.BlockSpec((1, d), lambda i: (0, 0)),
                  pl.BlockSpec((1, d), lambda i: (0, 0)),
                  pl.BlockSpec((1, d), lambda i: (0, 0)),
                  pl.BlockSpec((1, d), lambda i: (0, 0))],
        out_specs=[pl.BlockSpec((blk, dq), lambda i: (i, 0))] * NQ,
        out_shape=[jax.ShapeDtypeStruct((n, dq), jnp.float32)] * NQ,
    )(x, s, q, g, b)


def _sc_agg_build(n, e, dh):
    K = 64
    EW = e // NS
    NCH = EW // K
    ZBR = 8
    NZ = n // ZBR
    ZPER = -(-NZ // NS)
    mesh = plsc.VectorSubcoreMesh(core_axis_name="c", subcore_axis_name="s")

    @functools.partial(
        pl.kernel,
        mesh=mesh,
        out_type=[jax.ShapeDtypeStruct((n, dh), jnp.float32)] * NQ,
        scratch_types=[
            pltpu.VMEM((K,), jnp.int32),
            pltpu.VMEM((K,), jnp.int32),
            pltpu.VMEM((K, dh), jnp.float32),
            pltpu.VMEM((K,), jnp.int32),
            pltpu.VMEM((K,), jnp.int32),
            pltpu.VMEM((K, dh), jnp.float32),
            pltpu.VMEM((ZBR, dh), jnp.float32),
            pltpu.VMEM((ZBR, dh), jnp.float32),
            pltpu.VMEM_SHARED((n + 8, dh), jnp.float32),
            pltpu.SemaphoreType.DMA,
            pltpu.SemaphoreType.DMA,
        ],
    )
    def sc_agg(h0, h1, src, dst, agg0, agg1,
               src_v0, dst_v0, rows_v0, src_v1, dst_v1, rows_v1,
               zero_v, bounce_v, acc, sem0, sem1):
        cid = lax.axis_index("c")
        sid = lax.axis_index("s")
        z16f = jnp.zeros((16,), jnp.float32)

        def zrow(i, _):
            r = i // (dh // 16)
            c16 = i % (dh // 16)
            zero_v[r, pl.ds(c16 * 16, 16)] = z16f
            return _
        lax.fori_loop(0, ZBR * (dh // 16), zrow, None)

        def zacc(t, _):
            j = sid + t * NS

            @pl.when(j < NZ)
            def _():
                pltpu.sync_copy(zero_v, acc.at[pl.ds(j * ZBR, ZBR)])
            return _
        lax.fori_loop(0, ZPER, zacc, None)
        plsc.subcore_barrier()

        def scan_edges(h_ref):
            def prime(c, sv, dv, rv, sem):
                base = sid * EW + c * K
                pltpu.sync_copy(src.at[pl.ds(base, K)], sv)
                pltpu.sync_copy(dst.at[pl.ds(base, K)], dv)
                pltpu.async_copy(h_ref.at[sv], rv, sem)

            prime(0, src_v0, dst_v0, rows_v0, sem0)

            def pair(c2, _):
                c0 = c2 * 2

                prime(c0 + 1, src_v1, dst_v1, rows_v1, sem1)
                pltpu.make_async_copy(h_ref.at[src_v0], rows_v0, sem0).wait()
                pltpu.sync_copy(rows_v0, acc.at[dst_v0], add=True)

                @pl.when(c0 + 2 < NCH)
                def _():
                    prime(c0 + 2, src_v0, dst_v0, rows_v0, sem0)
                pltpu.make_async_copy(h_ref.at[src_v1], rows_v1, sem1).wait()
                pltpu.sync_copy(rows_v1, acc.at[dst_v1], add=True)
                return _
            lax.fori_loop(0, NCH // 2, pair, None)

        @pl.when(cid == 0)
        def _():
            scan_edges(h0)

        @pl.when(cid == 1)
        def _():
            scan_edges(h1)

        plsc.subcore_barrier()

        def flush(agg_ref):
            def go(t, _):
                j = sid + t * NS

                @pl.when(j < NZ)
                def _():
                    pltpu.sync_copy(acc.at[pl.ds(j * ZBR, ZBR)], bounce_v)
                    pltpu.sync_copy(bounce_v, agg_ref.at[pl.ds(j * ZBR, ZBR)])
                return _
            lax.fori_loop(0, ZPER, go, None)

        @pl.when(cid == 0)
        def _():
            flush(agg0)

        @pl.when(cid == 1)
        def _():
            flush(agg1)

    return sc_agg


def _hist_body(d_ref, out_ref, *, ngrid, recip):
    i = pl.program_id(0)
    d = d_ref[0]
    cols = lax.broadcasted_iota(jnp.int32, (d.shape[0], 128), 1)
    hi = lax.shift_right_logical(d, 7)
    lo = jnp.bitwise_and(d, 127)
    a = (hi == cols).astype(jnp.float32)
    b = (lo == cols).astype(jnp.float32)
    m = lax.dot_general(a, b, (((0,), (0,)), ((), ())),
                        preferred_element_type=jnp.float32)

    @pl.when(i == 0)
    def _():
        out_ref[...] = m

    @pl.when(i > 0)
    def _():
        out_ref[...] += m

    if recip:
        @pl.when(i == ngrid - 1)
        def _():
            out_ref[...] = 1.0 / jnp.maximum(out_ref[...], 1.0)


def _hist(idx, blk, recip):
    (e,) = idx.shape
    ngrid = e // blk
    idx3 = idx.reshape(ngrid, blk, 1)
    return pl.pallas_call(
        functools.partial(_hist_body, ngrid=ngrid, recip=recip),
        grid=(ngrid,),
        in_specs=[pl.BlockSpec((1, blk, 1), lambda i: (i, 0, 0))],
        out_specs=pl.BlockSpec((128, 128), lambda i: (0, 0)),
        out_shape=jax.ShapeDtypeStruct((128, 128), jnp.float32),
    )(idx3)


def _mlp_tail(xq, w0_ref, b0_ref, w1_ref, b1_ref, w2_ref, b2_ref,
              out_ref, s_ref, q_ref):
    i = pl.program_id(0)
    w0 = w0_ref[...]
    dq = xq[0].shape[1]
    y = b0_ref[...].astype(jnp.float32)
    for j, x in enumerate(xq):
        y = y + jnp.dot(x.astype(jnp.bfloat16), w0[j * dq:(j + 1) * dq, :],
                        preferred_element_type=jnp.float32)
    y = jnp.maximum(y, 0.0)
    y = jnp.dot(y.astype(jnp.bfloat16), w1_ref[...],
                preferred_element_type=jnp.float32) + b1_ref[...]
    y = jnp.maximum(y, 0.0)
    y = jnp.dot(y.astype(jnp.bfloat16), w2_ref[...],
                preferred_element_type=jnp.float32) + b2_ref[...]
    y = jnp.maximum(y, 0.0)
    out_ref[...] = y
    s = jnp.sum(y, axis=0, keepdims=True)
    q = jnp.sum(y * y, axis=0, keepdims=True)

    @pl.when(i == 0)
    def _():
        s_ref[...] = s
        q_ref[...] = q

    @pl.when(i > 0)
    def _():
        s_ref[...] += s
        q_ref[...] += q


def _bn_affine(fs_ref, fq_ref, g_ref, b_ref, n):
    mu = fs_ref[...] / n
    var = fq_ref[...] / n - mu * mu
    sc = g_ref[...] * lax.rsqrt(var + EPS)
    return sc, b_ref[...] - mu * sc


def _mlp_hom_body(a0, a1, dg_ref, fs_ref, fq_ref, g_ref, b_ref, w0, b0,
                  w1, b1, w2, b2, out_ref, s_ref, q_ref, *, n):
    dg = dg_ref[...]
    di = 1.0 / jnp.maximum(dg, 1.0)
    ind = (dg > 0.0).astype(jnp.float32)
    sc, sh = _bn_affine(fs_ref, fq_ref, g_ref, b_ref, n)
    dh = a0[...].shape[1]
    xq = [a[...] * di * sc[:, j * dh:(j + 1) * dh]
          + ind * sh[:, j * dh:(j + 1) * dh]
          for j, a in enumerate((a0, a1))]
    _mlp_tail(xq, w0, b0, w1, b1, w2, b2, out_ref, s_ref, q_ref)


def _mlp_het_body(f0, f1, a0, a1, dg_ref, fs_ref, fq_ref, g_ref, b_ref,
                  w0, b0, w1, b1, w2, b2, out_ref, s_ref, q_ref, *, n):
    dg = dg_ref[...]
    di = 1.0 / jnp.maximum(dg, 1.0)
    ind = (dg > 0.0).astype(jnp.float32)
    sc, sh = _bn_affine(fs_ref, fq_ref, g_ref, b_ref, n)
    dh = a0[...].shape[1]
    xq = [f[...] - (a[...] * di * sc[:, j * dh:(j + 1) * dh]
                    + ind * sh[:, j * dh:(j + 1) * dh])
          for j, (f, a) in enumerate(zip((f0, f1), (a0, a1)))]
    _mlp_tail(xq, w0, b0, w1, b1, w2, b2, out_ref, s_ref, q_ref)


def _mlp_single_body(h0, h1, w0, b0, w1, b1, w2, b2,
                     out_ref, s_ref, q_ref):
    _mlp_tail([h0[...], h1[...]],
              w0, b0, w1, b1, w2, b2, out_ref, s_ref, q_ref)


def _mlp_call(body, row_args, row_specs, deg_col, bnin, p, blk):
    n = row_args[0].shape[0]
    h = p["W1"].shape[0]
    args = list(row_args)
    specs = list(row_specs)
    if deg_col is not None:
        body = functools.partial(body, n=float(n))
        args.append(deg_col)
        specs.append(pl.BlockSpec((blk, 1), lambda i: (i, 0)))
        d = bnin[0].shape[1]
        args += list(bnin)
        specs += [pl.BlockSpec((1, d), lambda i: (0, 0))] * 4
    d0 = p["W0"].shape[0]
    bf = jnp.bfloat16
    args += [p["W0"].astype(bf), p["b0"].reshape(1, h),
             p["W1"].astype(bf), p["b1"].reshape(1, h),
             p["W2"].astype(bf), p["b2"].reshape(1, h)]
    specs += [pl.BlockSpec((d0, h), lambda i: (0, 0)),
              pl.BlockSpec((1, h), lambda i: (0, 0)),
              pl.BlockSpec((h, h), lambda i: (0, 0)),
              pl.BlockSpec((1, h), lambda i: (0, 0)),
              pl.BlockSpec((h, h), lambda i: (0, 0)),
              pl.BlockSpec((1, h), lambda i: (0, 0))]
    return pl.pallas_call(
        body,
        grid=(n // blk,),
        in_specs=specs,
        out_specs=[pl.BlockSpec((blk, h), lambda i: (i, 0)),
                   pl.BlockSpec((1, h), lambda i: (0, 0)),
                   pl.BlockSpec((1, h), lambda i: (0, 0))],
        out_shape=[jax.ShapeDtypeStruct((n, h), jnp.float32),
                   jax.ShapeDtypeStruct((1, h), jnp.float32),
                   jax.ShapeDtypeStruct((1, h), jnp.float32)],
    )(*args)


def _combine_body(lh_ref, le_ref, ls_ref, sh_ref, qh_ref, se_ref, qe_ref,
                  ss_ref, qs_ref, gh_ref, bh_ref, ge_ref, be_ref, gs_ref,
                  bs_ref, f_ref, gw_ref, gb_ref, fw_ref, fb_ref, t_ref,
                  lab_ref, cnt_ref, logit_ref, kl_ref, ce_ref, s0_ref,
                  s1_ref, s2_ref, loss_ref, *, n, nt, ngrid):
    i = pl.program_id(0)

    def norm(l_ref, s_ref, q_ref, g_ref, b_ref):
        mu = s_ref[...] / n
        var = q_ref[...] / n - mu * mu
        sc = g_ref[...] * lax.rsqrt(var + EPS)
        return l_ref[...] * sc + (b_ref[...] - mu * sc)

    xh = norm(lh_ref, sh_ref, qh_ref, gh_ref, bh_ref)
    xe = norm(le_ref, se_ref, qe_ref, ge_ref, be_ref)
    xs = norm(ls_ref, ss_ref, qs_ref, gs_ref, bs_ref)

    z = (jnp.dot(f_ref[...], gw_ref[...], preferred_element_type=jnp.float32)
         + gb_ref[...])
    zm = jnp.max(z, axis=1, keepdims=True)
    ez = jnp.exp(z - zm)
    cf = ez / jnp.sum(ez, axis=1, keepdims=True)
    c0 = cf[:, 0:1]
    c1 = cf[:, 1:2]
    c2 = cf[:, 2:3]
    m = jnp.minimum(c0, jnp.minimum(c1, c2))
    e2 = c2 <= m
    e1 = jnp.logical_and(jnp.logical_not(e2), c1 <= m)
    e0 = jnp.logical_and(jnp.logical_not(e2), jnp.logical_not(e1))
    mk0 = jnp.where(e0, c0 * -100000.0, c0)
    mk1 = jnp.where(e1, c1 * -100000.0, c1)
    mk2 = jnp.where(e2, c2 * -100000.0, c2)
    mx = jnp.maximum(mk0, jnp.maximum(mk1, mk2))
    x0 = jnp.exp(mk0 - mx)
    x1 = jnp.exp(mk1 - mx)
    x2 = jnp.exp(mk2 - mx)
    zs = x0 + x1 + x2
    w0 = x0 / zs
    w1 = x1 / zs
    w2 = x2 / zs

    emb = w0 * xh + w1 * xe + w2 * xs
    lr = (jnp.dot(emb, fw_ref[...], preferred_element_type=jnp.float32)
          + fb_ref[...])
    lm = jnp.max(lr, axis=1, keepdims=True)
    el = jnp.exp(lr - lm)
    sm = el / jnp.sum(el, axis=1, keepdims=True)
    logit_ref[...] = sm

    t = sm / TAU
    kl = jnp.sum(t * (jnp.log(t) - t_ref[...] / TAU)).reshape(1, 1)

    mx2 = jnp.max(sm, axis=1, keepdims=True)
    logp = sm - mx2 - jnp.log(jnp.sum(jnp.exp(sm - mx2), axis=1, keepdims=True))
    lanes = lax.broadcasted_iota(jnp.int32, sm.shape, 1)
    pick = jnp.sum(jnp.where(lanes == lab_ref[...], logp, 0.0), axis=1,
                   keepdims=True)
    ce = (-jnp.sum(cnt_ref[...] * pick)).reshape(1, 1)

    s0 = jnp.sum(w0).reshape(1, 1)
    s1 = jnp.sum(w1).reshape(1, 1)
    s2 = jnp.sum(w2).reshape(1, 1)

    @pl.when(i == 0)
    def _():
        kl_ref[...] = kl
        ce_ref[...] = ce
        s0_ref[...] = s0
        s1_ref[...] = s1
        s2_ref[...] = s2

    @pl.when(i > 0)
    def _():
        kl_ref[...] += kl
        ce_ref[...] += ce
        s0_ref[...] += s0
        s1_ref[...] += s1
        s2_ref[...] += s2

    @pl.when(i == ngrid - 1)
    def _():
        loss1 = kl_ref[...] / n
        loss2 = ce_ref[...] / nt
        third = 1.0 / 3.0
        aux = (jnp.abs(s0_ref[...] / n - third)
               + jnp.abs(s1_ref[...] / n - third)
               + jnp.abs(s2_ref[...] / n - third))
        loss_ref[...] = (LAMBDA1 * loss1 + (1.0 - LAMBDA1) * loss2
                         + LAMBDA2 * aux)


def _combine(lh, le, ls, stats, bns, feature, gwp, gbp, fw, fb, teacher,
             lab_col, cnt_col, nt, blk):
    n, h = lh.shape
    c = fw.shape[1]
    d = feature.shape[1]
    ngrid = n // blk
    sh, qh, se, qe, ss, qs = stats
    gh, bh, ge, be, gs, bs = bns
    row = lambda w: pl.BlockSpec((blk, w), lambda i: (i, 0))
    one = lambda w: pl.BlockSpec((1, w), lambda i: (0, 0))
    scl = lambda: pl.BlockSpec((1, 1), lambda i: (0, 0))
    return pl.pallas_call(
        functools.partial(_combine_body, n=float(n), nt=float(nt),
                          ngrid=ngrid),
        grid=(ngrid,),
        in_specs=[row(h), row(h), row(h),
                  one(h), one(h), one(h), one(h), one(h), one(h),
                  one(h), one(h), one(h), one(h), one(h), one(h),
                  row(d), pl.BlockSpec((d, 128), lambda i: (0, 0)), one(128),
                  pl.BlockSpec((h, c), lambda i: (0, 0)), one(c),
                  row(c), row(1), row(1)],
        out_specs=[row(c), scl(), scl(), scl(), scl(), scl(), scl()],
        out_shape=[jax.ShapeDtypeStruct((n, c), jnp.float32)]
                  + [jax.ShapeDtypeStruct((1, 1), jnp.float32)] * 6,
    )(lh, le, ls, sh, qh, se, qe, ss, qs, gh, bh, ge, be, gs, bs,
      feature, gwp, gbp, fw, fb, teacher, lab_col, cnt_col)


def kernel(feature, edge_index, label, train_nodes, teacher_logit, params):
    n, d = feature.shape
    e = edge_index.shape[1]
    nt = train_nodes.shape[0]
    h = params["fin_W"].shape[0]
    c = params["fin_W"].shape[1]
    dq = d // NQ

    fsum, fsq = _colstats(feature, 2000)
    hq = _bnapply(feature, fsum, fsq,
                  params["bn_in_g"].reshape(1, d),
                  params["bn_in_b"].reshape(1, d), 2000)

    KCH = 4096
    ep = -(-e // KCH) * KCH
    src_p = jnp.concatenate(
        [edge_index[0], jnp.zeros((ep - e,), jnp.int32)])
    dst_p = jnp.concatenate(
        [edge_index[1], jnp.full((ep - e,), n, jnp.int32)])
    ntp = -(-nt // 128) * 128
    tn_p = jnp.concatenate([train_nodes, jnp.full((ntp - nt,), n, jnp.int32)])

    fq = [feature[:, j * dq:(j + 1) * dq] for j in range(NQ)]
    sc_agg = _sc_agg_build(n, ep, dq)
    agg0, agg1 = sc_agg(fq[0], fq[1], src_p, dst_p)
    aggq = [agg0, agg1]
    deg_col = _hist(dst_p, 4096, False).reshape(16384, 1)[:n]
    cnt_col = _hist(tn_p, ntp, False).reshape(16384, 1)[:n]

    MB = 1000
    aspec = [pl.BlockSpec((MB, dq), lambda i: (i, 0)) for _ in range(NQ)]
    bnin = (fsum, fsq, params["bn_in_g"].reshape(1, d),
            params["bn_in_b"].reshape(1, d))

    lh, sh_, qh_ = _mlp_call(_mlp_hom_body, aggq, aspec, deg_col, bnin,
                             params["hom"], MB)
    le, se_, qe_ = _mlp_call(_mlp_het_body, fq + aggq,
                             aspec + aspec, deg_col, bnin, params["het"], MB)
    ls, ss_, qs_ = _mlp_call(_mlp_single_body, list(hq), aspec, None, None,
                             params["single"], MB)

    gwp = jnp.zeros((d, 128), jnp.float32).at[:, :3].set(params["gate_W"])
    gbp = jnp.full((1, 128), -1e30, jnp.float32).at[0, :3].set(params["gate_b"])
    stats = (sh_, qh_, se_, qe_, ss_, qs_)
    bns = (params["hom"]["bn_g"].reshape(1, h), params["hom"]["bn_b"].reshape(1, h),
           params["het"]["bn_g"].reshape(1, h), params["het"]["bn_b"].reshape(1, h),
           params["single"]["bn_g"].reshape(1, h), params["single"]["bn_b"].reshape(1, h))
    logit, _kl, _ce, _s0, _s1, _s2, loss = _combine(
        lh, le, ls, stats, bns, feature, gwp, gbp,
        params["fin_W"], params["fin_b"].reshape(1, c), teacher_logit,
        label.reshape(n, 1), cnt_col, nt, 400)
    return logit, loss.reshape(())
